# baseline (speedup 1.0000x reference)
import functools

import jax
import jax.numpy as jnp
from jax import lax
from jax.experimental import pallas as pl
from jax.experimental.pallas import tpu as pltpu
from jax.experimental.pallas import tpu_sc as plsc

NC = 2
NS = 16
NW = NC * NS
K = 128


def _pad2(m, rows, cols):
    return jnp.pad(m, ((0, rows - m.shape[0]), (0, cols - m.shape[1])))


def _sc_gather_multi(pairs):
    N = pairs[0][1].shape[0]
    n_chunks = N // K
    iters = (n_chunks + NW - 1) // NW
    n_t = len(pairs)
    mesh = plsc.VectorSubcoreMesh(core_axis_name="c", subcore_axis_name="s")

    scratch = []
    for t, _ in pairs:
        scratch.append(pltpu.VMEM((K,), jnp.int32))
        scratch.append(pltpu.VMEM((K, t.shape[1]), jnp.float32))
    scratch.append(pltpu.SemaphoreType.DMA)

    @functools.partial(
        pl.kernel,
        mesh=mesh,
        out_type=[jax.ShapeDtypeStruct((N, t.shape[1]), jnp.float32)
                  for t, _ in pairs],
        scratch_types=scratch,
    )
    def k(*refs):
        tabs = refs[0:2 * n_t:2]
        idxs = refs[1:2 * n_t:2]
        outs = refs[2 * n_t:3 * n_t]
        bufs = refs[3 * n_t:3 * n_t + 2 * n_t]
        sem = refs[-1]
        wid = lax.axis_index("s") * NC + lax.axis_index("c")

        def body(i, carry):
            cid = i * NW + wid

            @pl.when(cid < n_chunks)
            def _():
                b = cid * K
                for j in range(n_t):
                    idx_v, rows_v = bufs[2 * j], bufs[2 * j + 1]
                    pltpu.sync_copy(idxs[j].at[pl.ds(b, K)], idx_v)
                    pltpu.async_copy(tabs[j].at[idx_v], rows_v, sem).wait()
                    pltpu.sync_copy(rows_v, outs[j].at[pl.ds(b, K)])

            return carry

        lax.fori_loop(0, iters, body, 0)

    args = []
    for t, ix in pairs:
        args += [t, ix]
    res = k(*args)
    return list(res) if isinstance(res, (list, tuple)) else [res]


def _sc_scatter_multi(vals_list, idx, S):
    N = idx.shape[0]
    n_chunks = N // K
    iters = (n_chunks + NS - 1) // NS
    n_v = len(vals_list)
    zeros = jnp.zeros((S, 128), jnp.float32)
    mesh = plsc.VectorSubcoreMesh(core_axis_name="c", subcore_axis_name="s")
    phases = []
    for v, a in enumerate(vals_list):
        for cb in range(0, a.shape[1], 256):
            phases.append((v, cb))
    r_lo, r_hi = (S // 16) // 8 * 8, S - 15 * ((S // 16) // 8 * 8)

    @functools.partial(
        pl.kernel,
        mesh=mesh,
        out_type=[jax.ShapeDtypeStruct((S, a.shape[1]), jnp.float32)
                  for a in vals_list],
        scratch_types=[
            pltpu.VMEM((K,), jnp.int32),
            pltpu.VMEM((K, 128), jnp.float32),
            pltpu.VMEM_SHARED((S, 128), jnp.float32),
        ],
    )
    def k(*refs):
        vals = refs[0:n_v]
        idx_hbm = refs[n_v]
        zeros_hbm = refs[n_v + 1]
        outs = refs[n_v + 2:n_v + 2 + n_v]
        idx_v, val_v, acc = refs[-3:]
        c = lax.axis_index("c")
        s = lax.axis_index("s")
        r0 = s * r_lo

        for v, cb in phases:
            D = vals_list[v].shape[1]
            col = cb + c * 128
            active = col < D

            @pl.when(active & (s < 15))
            def _():
                pltpu.sync_copy(zeros_hbm.at[pl.ds(r0, r_lo)],
                                acc.at[pl.ds(r0, r_lo)])

            @pl.when(active & (s == 15))
            def _():
                pltpu.sync_copy(zeros_hbm.at[pl.ds(15 * r_lo, r_hi)],
                                acc.at[pl.ds(15 * r_lo, r_hi)])

            plsc.subcore_barrier()

            def body(i, carry):
                cid = i * NS + s

                @pl.when(active & (cid < n_chunks))
                def _():
                    b = cid * K
                    pltpu.sync_copy(idx_hbm.at[pl.ds(b, K)], idx_v)
                    pltpu.sync_copy(vals[v].at[pl.ds(b, K), pl.ds(col, 128)],
                                    val_v)
                    pltpu.sync_copy(val_v, acc.at[idx_v], add=True)

                return carry

            lax.fori_loop(0, iters, body, 0)
            plsc.subcore_barrier()

            @pl.when(active & (s < 15))
            def _():
                pltpu.sync_copy(acc.at[pl.ds(r0, r_lo)],
                                outs[v].at[pl.ds(r0, r_lo), pl.ds(col, 128)])

            @pl.when(active & (s == 15))
            def _():
                pltpu.sync_copy(acc.at[pl.ds(15 * r_lo, r_hi)],
                                outs[v].at[pl.ds(15 * r_lo, r_hi), pl.ds(col, 128)])

            plsc.subcore_barrier()

    res = k(*vals_list, idx, zeros)
    return list(res) if isinstance(res, (list, tuple)) else [res]


def _mlp_refs(x, wrefs):
    h = x * wrefs[0][...]
    for w in wrefs[1:]:
        h = jnp.maximum(h, 0.0)
        h = jnp.dot(h, w[...], preferred_element_type=jnp.float32)
    return h


def _full(shape):
    return pl.BlockSpec(shape, lambda i: tuple(0 for _ in shape))


def _blk(be, d):
    return pl.BlockSpec((be, d), lambda i: (i, 0))


def _tc_edge(emb, sh, g, ws, Bm, be):
    N, D = g.shape
    nw = len(ws)

    def body(*refs):
        emb_ref, sh_ref, g_ref = refs[:3]
        wrefs = refs[3:3 + nw]
        B_ref = refs[3 + nw]
        out_ref = refs[3 + nw + 1]
        w = _mlp_refs(emb_ref[...], wrefs)
        shB = jnp.dot(sh_ref[...], B_ref[...], preferred_element_type=jnp.float32)
        out_ref[...] = w * g_ref[...] * shB

    return pl.pallas_call(
        body,
        grid=(N // be,),
        in_specs=[_blk(be, 1), _blk(be, 9), _blk(be, D)]
        + [_full(w.shape) for w in ws] + [_full(Bm.shape)],
        out_specs=_blk(be, D),
        out_shape=jax.ShapeDtypeStruct((N, D), jnp.float32),
    )(emb, sh, g, *ws, Bm)


def _tc_node_mm(x, mats, br):
    S, Din = x.shape

    def body(*refs):
        x_ref = refs[0]
        m_refs = refs[1:1 + len(mats)]
        out_refs = refs[1 + len(mats):]
        xv = x_ref[...]
        for m_ref, o_ref in zip(m_refs, out_refs):
            o_ref[...] = jnp.dot(xv, m_ref[...], preferred_element_type=jnp.float32)

    return pl.pallas_call(
        body,
        grid=(S // br,),
        in_specs=[_blk(br, Din)] + [_full(m.shape) for m in mats],
        out_specs=[_blk(br, m.shape[1]) for m in mats],
        out_shape=[jax.ShapeDtypeStruct((S, m.shape[1]), jnp.float32) for m in mats],
    )(x, *mats)


def _tc_edge_ch(emb, sh, gC, gH, wsC, wsH, BC, BH, be):
    N = gC.shape[0]
    DC, DH = gC.shape[1], gH.shape[1]

    def body(*refs):
        emb_ref, sh_ref, gC_ref, gH_ref = refs[:4]
        i = 4
        wC = refs[i:i + 4]; i += 4
        wH = refs[i:i + 4]; i += 4
        BC_ref, BH_ref, outC_ref, outH_ref = refs[i:i + 4]
        shv = sh_ref[...]
        ev = emb_ref[...]
        mC = _mlp_refs(ev, wC)
        mH = _mlp_refs(ev, wH)
        shBC = jnp.dot(shv, BC_ref[...], preferred_element_type=jnp.float32)
        shBH = jnp.dot(shv, BH_ref[...], preferred_element_type=jnp.float32)
        outC_ref[...] = mC * gC_ref[...] * shBC
        outH_ref[...] = mH * gH_ref[...] * shBH

    return pl.pallas_call(
        body,
        grid=(N // be,),
        in_specs=[_blk(be, 1), _blk(be, 9), _blk(be, DC), _blk(be, DH)]
        + [_full(w.shape) for w in wsC] + [_full(w.shape) for w in wsH]
        + [_full(BC.shape), _full(BH.shape)],
        out_specs=[_blk(be, DC), _blk(be, DH)],
        out_shape=[jax.ShapeDtypeStruct((N, DC), jnp.float32),
                   jax.ShapeDtypeStruct((N, DH), jnp.float32)],
    )(emb, sh, gC, gH, *wsC, *wsH, BC, BH)


def _tc_bond(emb_b, sh_b, ga, gb, ws_bond, ws_x, Ax, Bx, blk0, nblk, be):
    Dx = Ax.shape[1]
    nb, nx = len(ws_bond), len(ws_x)

    def off(d):
        return pl.BlockSpec((be, d), lambda i: (blk0 + i, 0))

    def body(*refs):
        emb_ref, sh_ref, ga_ref, gb_ref = refs[:4]
        i = 4
        wb = refs[i:i + nb]; i += nb
        wx = refs[i:i + nx]; i += nx
        A_ref, B_ref, out_ref = refs[i:i + 3]
        ev = emb_ref[...]
        hf = _mlp_refs(ev, wb) * ga_ref[...] * gb_ref[...]
        mx = _mlp_refs(ev, wx)
        hA = jnp.dot(hf, A_ref[...], preferred_element_type=jnp.float32)
        sB = jnp.dot(sh_ref[...], B_ref[...], preferred_element_type=jnp.float32)
        out_ref[...] = mx * hA * sB

    return pl.pallas_call(
        body,
        grid=(nblk,),
        in_specs=[off(1), off(9), off(ga.shape[1]), off(gb.shape[1])]
        + [_full(w.shape) for w in ws_bond] + [_full(w.shape) for w in ws_x]
        + [_full(Ax.shape), _full(Bx.shape)],
        out_specs=_blk(be, Dx),
        out_shape=jax.ShapeDtypeStruct((nblk * be, Dx), jnp.float32),
    )(emb_b, sh_b, ga, gb, *ws_bond, *ws_x, Ax, Bx)


def kernel(sh, emb, f_in, edge_src, edge_dst, num_nodes, num_neighbors,
           HH_ind, CC_ind, CH_ind, fc1, fc2, fc_bond, fcHH, fcCC, fcCH, fcC, fcH,
           A1, B1, A2, B2, Ab, Bb, AHH, BHH, ACC, BCC, ACH, BCH, AC, BC, AH, BH):
    S = f_in.shape[0]
    E = sh.shape[0]
    NB = HH_ind.shape[0]

    inv = 1.0 / jnp.sqrt(jnp.asarray(num_neighbors, jnp.float32))
    w1l = _pad2(fc1[-1] * inv, 16, 128)
    w2l = _pad2(fc2[-1] * inv, 16, 256)
    wCl = _pad2(fcC[-1] * inv, 16, 512)
    wHl = _pad2(fcH[-1] * inv, 16, 128)

    (p1,) = _tc_node_mm(f_in, [_pad2(A1, 2, 128)], 2000)
    (g1,) = _sc_gather_multi([(p1, edge_src)])
    ef1 = _tc_edge(emb, sh, g1, list(fc1[:-1]) + [w1l], _pad2(B1, 9, 128), 1600)
    (nf1,) = _sc_scatter_multi([ef1], edge_dst, S)

    (p2,) = _tc_node_mm(nf1, [_pad2(A2, 128, 256)], 2000)
    (g2,) = _sc_gather_multi([(p2, edge_src)])
    ef2 = _tc_edge(emb, sh, g2, list(fc2[:-1]) + [w2l], _pad2(B2, 9, 256), 1600)
    (nf2,) = _sc_scatter_multi([ef2], edge_dst, S)

    TA, TB, TCt, THt = _tc_node_mm(
        nf2,
        [_pad2(Ab, 256, 256), _pad2(Bb, 256, 256),
         _pad2(AC, 256, 512), _pad2(AH, 256, 128)], 2000)

    srcf = edge_src.astype(jnp.float32).reshape(E, 1)
    dstf = edge_dst.astype(jnp.float32).reshape(E, 1)
    meta = jnp.pad(jnp.concatenate([srcf, dstf, emb, sh], axis=1),
                   ((0, 0), (0, 116)))
    inds = jnp.concatenate(
        [HH_ind, CC_ind, CH_ind, jnp.zeros((E - 3 * NB,), jnp.int32)])

    gC, gH, metag = _sc_gather_multi(
        [(TCt, edge_src), (THt, edge_src), (meta, inds)])
    edge_C, edge_H = _tc_edge_ch(
        emb, sh, gC, gH,
        list(fcC[:-1]) + [wCl], list(fcH[:-1]) + [wHl],
        _pad2(BC, 9, 512), _pad2(BH, 9, 128), 1600)
    node_C, node_H = _sc_scatter_multi([edge_C, edge_H], edge_dst, S)

    NBP = 122880
    b_src = metag[:NBP, 0].astype(jnp.int32)
    b_dst = metag[:NBP, 1].astype(jnp.int32)
    emb_b = metag[:NBP, 2:3]
    sh_b = metag[:NBP, 3:12]
    gA, gB = _sc_gather_multi([(TA, b_src), (TB, b_dst)])

    be = 2000
    nblk = NB // be
    wsb = list(fc_bond[:-1]) + [_pad2(fc_bond[-1], 16, 256)]
    edge_HH = _tc_bond(emb_b, sh_b, gA, gB, wsb, fcHH,
                       _pad2(AHH, 256, 50), BHH, 0 * nblk, nblk, be)
    edge_CC = _tc_bond(emb_b, sh_b, gA, gB, wsb, fcCC,
                       _pad2(ACC, 256, 392), BCC, 1 * nblk, nblk, be)
    edge_CH = _tc_bond(emb_b, sh_b, gA, gB, wsb, fcCH,
                       _pad2(ACH, 256, 140), BCH, 2 * nblk, nblk, be)

    hC, hH = 196, 25
    return (node_H[:, :hH], node_C[:, :hC], edge_HH[:, :hH], edge_CH[:, :70],
            edge_CC[:, :hC],
            node_H[:, hH:2 * hH], node_C[:, hC:2 * hC], edge_HH[:, hH:2 * hH],
            edge_CH[:, 70:140], edge_CC[:, hC:2 * hC])


# speedup vs baseline: 1.2049x; 1.2049x over previous
import functools

import jax
import jax.numpy as jnp
from jax import lax
from jax.experimental import pallas as pl
from jax.experimental.pallas import tpu as pltpu
from jax.experimental.pallas import tpu_sc as plsc

NC = 2
NS = 16
NW = NC * NS
K = 128


def _pad2(m, rows, cols):
    return jnp.pad(m, ((0, rows - m.shape[0]), (0, cols - m.shape[1])))


def _sc_gather_multi(pairs, K=128, nbuf=2):
    N = pairs[0][1].shape[0]
    n_chunks = N // K
    iters = (n_chunks + NW - 1) // NW
    rounds = (iters + nbuf - 1) // nbuf
    n_t = len(pairs)
    uidx, uslot = [], []
    for _, ix in pairs:
        for u, ux in enumerate(uidx):
            if ux is ix:
                uslot.append(u)
                break
        else:
            uslot.append(len(uidx))
            uidx.append(ix)
    n_u = len(uidx)
    mesh = plsc.VectorSubcoreMesh(core_axis_name="c", subcore_axis_name="s")

    scratch = (
        [pltpu.VMEM((K,), jnp.int32) for _ in range(n_u * nbuf)]
        + [pltpu.VMEM((K, t.shape[1]), jnp.float32)
           for t, _ in pairs for _ in range(nbuf)]
        + [pltpu.SemaphoreType.DMA for _ in range(2 * nbuf)]
    )

    @functools.partial(
        pl.kernel,
        mesh=mesh,
        out_type=[jax.ShapeDtypeStruct((N, t.shape[1]), jnp.float32)
                  for t, _ in pairs],
        scratch_types=scratch,
    )
    def k(*refs):
        tabs = refs[0:n_t]
        idxs = refs[n_t:n_t + n_u]
        outs = refs[n_t + n_u:n_t + n_u + n_t]
        sc = refs[n_t + n_u + n_t:]
        ib = [sc[u * nbuf:(u + 1) * nbuf] for u in range(n_u)]
        sc = sc[n_u * nbuf:]
        rb = [sc[j * nbuf:(j + 1) * nbuf] for j in range(n_t)]
        sc = sc[n_t * nbuf:]
        gsem, ssem = sc[:nbuf], sc[nbuf:2 * nbuf]
        wid = lax.axis_index("s") * NC + lax.axis_index("c")

        def gath(j, b):
            return pltpu.make_async_copy(
                tabs[j].at[ib[uslot[j]][b]], rb[j][b], gsem[b])

        def stor(j, b, off):
            return pltpu.make_async_copy(
                rb[j][b], outs[j].at[pl.ds(off, K)], ssem[b])

        def body(r, carry):
            cs = []
            for b in range(nbuf):
                cid = (r * nbuf + b) * NW + wid
                cs.append((cid < n_chunks, cid * K))
            for b in range(nbuf):
                pred, off = cs[b]

                @pl.when(pred)
                def _(b=b, off=off):
                    for u in range(n_u):
                        pltpu.sync_copy(idxs[u].at[pl.ds(off, K)], ib[u][b])
                    for j in range(n_t):
                        gath(j, b).start()

            for b in range(nbuf):
                pred, off = cs[b]

                @pl.when(pred)
                def _(b=b, off=off):
                    for j in range(n_t):
                        gath(j, b).wait()
                    for j in range(n_t):
                        stor(j, b, off).start()

            for b in range(nbuf):
                pred, off = cs[b]

                @pl.when(pred)
                def _(b=b, off=off):
                    for j in range(n_t):
                        stor(j, b, off).wait()

            return carry

        lax.fori_loop(0, rounds, body, 0)

    res = k(*[t for t, _ in pairs], *uidx)
    return list(res) if isinstance(res, (list, tuple)) else [res]


def _sc_scatter_multi(vals_list, idx, S):
    N = idx.shape[0]
    n_chunks = N // K
    iters = (n_chunks + NS - 1) // NS
    n_v = len(vals_list)
    zeros = jnp.zeros((S, 128), jnp.float32)
    mesh = plsc.VectorSubcoreMesh(core_axis_name="c", subcore_axis_name="s")
    phases = []
    for v, a in enumerate(vals_list):
        for cb in range(0, a.shape[1], 256):
            phases.append((v, cb))
    r_lo, r_hi = (S // 16) // 8 * 8, S - 15 * ((S // 16) // 8 * 8)

    nbuf = 2
    rounds = (iters + nbuf - 1) // nbuf

    @functools.partial(
        pl.kernel,
        mesh=mesh,
        out_type=[jax.ShapeDtypeStruct((S, a.shape[1]), jnp.float32)
                  for a in vals_list],
        scratch_types=(
            [pltpu.VMEM((K,), jnp.int32) for _ in range(nbuf)]
            + [pltpu.VMEM((K, 128), jnp.float32) for _ in range(nbuf)]
            + [pltpu.VMEM_SHARED((S, 128), jnp.float32)]
            + [pltpu.SemaphoreType.DMA for _ in range(2 * nbuf)]
        ),
    )
    def k(*refs):
        vals = refs[0:n_v]
        idx_hbm = refs[n_v]
        zeros_hbm = refs[n_v + 1]
        outs = refs[n_v + 2:n_v + 2 + n_v]
        sc = refs[n_v + 2 + n_v:]
        ib, vb = sc[:nbuf], sc[nbuf:2 * nbuf]
        acc = sc[2 * nbuf]
        lsem = sc[2 * nbuf + 1:2 * nbuf + 1 + nbuf]
        asem = sc[2 * nbuf + 1 + nbuf:2 * nbuf + 1 + 2 * nbuf]
        c = lax.axis_index("c")
        s = lax.axis_index("s")
        r0 = s * r_lo

        for v, cb in phases:
            D = vals_list[v].shape[1]
            col = cb + c * 128
            active = col < D

            @pl.when(active & (s < 15))
            def _():
                pltpu.sync_copy(zeros_hbm.at[pl.ds(r0, r_lo)],
                                acc.at[pl.ds(r0, r_lo)])

            @pl.when(active & (s == 15))
            def _():
                pltpu.sync_copy(zeros_hbm.at[pl.ds(15 * r_lo, r_hi)],
                                acc.at[pl.ds(15 * r_lo, r_hi)])

            plsc.subcore_barrier()

            def ldi(b, off):
                return pltpu.make_async_copy(
                    idx_hbm.at[pl.ds(off, K)], ib[b], lsem[b])

            def ldv(b, off):
                return pltpu.make_async_copy(
                    vals[v].at[pl.ds(off, K), pl.ds(col, 128)], vb[b], lsem[b])

            def addv(b):
                return pltpu.make_async_copy(vb[b], acc.at[ib[b]], asem[b])

            def body(i, carry):
                cs = []
                for b in range(nbuf):
                    cid = (i * nbuf + b) * NS + s
                    cs.append((active & (cid < n_chunks), cid * K))
                for b in range(nbuf):
                    pred, off = cs[b]

                    @pl.when(pred)
                    def _(b=b, off=off):
                        ldi(b, off).start()
                        ldv(b, off).start()

                for b in range(nbuf):
                    pred, off = cs[b]

                    @pl.when(pred)
                    def _(b=b, off=off):
                        ldi(b, off).wait()
                        ldv(b, off).wait()
                        addv(b).start(add=True)

                for b in range(nbuf):
                    pred, off = cs[b]

                    @pl.when(pred)
                    def _(b=b):
                        addv(b).wait()

                return carry

            lax.fori_loop(0, rounds, body, 0)
            plsc.subcore_barrier()

            @pl.when(active & (s < 15))
            def _():
                pltpu.sync_copy(acc.at[pl.ds(r0, r_lo)],
                                outs[v].at[pl.ds(r0, r_lo), pl.ds(col, 128)])

            @pl.when(active & (s == 15))
            def _():
                pltpu.sync_copy(acc.at[pl.ds(15 * r_lo, r_hi)],
                                outs[v].at[pl.ds(15 * r_lo, r_hi), pl.ds(col, 128)])

            plsc.subcore_barrier()

    res = k(*vals_list, idx, zeros)
    return list(res) if isinstance(res, (list, tuple)) else [res]


def _mlp_refs(x, wrefs):
    h = x * wrefs[0][...]
    for w in wrefs[1:]:
        h = jnp.maximum(h, 0.0)
        h = jnp.dot(h, w[...], preferred_element_type=jnp.float32)
    return h


def _full(shape):
    return pl.BlockSpec(shape, lambda i: tuple(0 for _ in shape))


def _blk(be, d):
    return pl.BlockSpec((be, d), lambda i: (i, 0))


def _tc_edge(emb, sh, g, ws, Bm, be):
    N, D = g.shape
    nw = len(ws)

    def body(*refs):
        emb_ref, sh_ref, g_ref = refs[:3]
        wrefs = refs[3:3 + nw]
        B_ref = refs[3 + nw]
        out_ref = refs[3 + nw + 1]
        w = _mlp_refs(emb_ref[...], wrefs)
        shB = jnp.dot(sh_ref[...], B_ref[...], preferred_element_type=jnp.float32)
        out_ref[...] = w * g_ref[...] * shB

    return pl.pallas_call(
        body,
        grid=(N // be,),
        in_specs=[_blk(be, 1), _blk(be, 9), _blk(be, D)]
        + [_full(w.shape) for w in ws] + [_full(Bm.shape)],
        out_specs=_blk(be, D),
        out_shape=jax.ShapeDtypeStruct((N, D), jnp.float32),
    )(emb, sh, g, *ws, Bm)


def _tc_node_mm(x, mats, br):
    S, Din = x.shape

    def body(*refs):
        x_ref = refs[0]
        m_refs = refs[1:1 + len(mats)]
        out_refs = refs[1 + len(mats):]
        xv = x_ref[...]
        for m_ref, o_ref in zip(m_refs, out_refs):
            o_ref[...] = jnp.dot(xv, m_ref[...], preferred_element_type=jnp.float32)

    return pl.pallas_call(
        body,
        grid=(S // br,),
        in_specs=[_blk(br, Din)] + [_full(m.shape) for m in mats],
        out_specs=[_blk(br, m.shape[1]) for m in mats],
        out_shape=[jax.ShapeDtypeStruct((S, m.shape[1]), jnp.float32) for m in mats],
    )(x, *mats)


def _tc_edge_ch(emb, sh, gC, gH, wsC, wsH, BC, BH, be):
    N = gC.shape[0]
    DC, DH = gC.shape[1], gH.shape[1]

    def body(*refs):
        emb_ref, sh_ref, gC_ref, gH_ref = refs[:4]
        i = 4
        wC = refs[i:i + 4]; i += 4
        wH = refs[i:i + 4]; i += 4
        BC_ref, BH_ref, outC_ref, outH_ref = refs[i:i + 4]
        shv = sh_ref[...]
        ev = emb_ref[...]
        mC = _mlp_refs(ev, wC)
        mH = _mlp_refs(ev, wH)
        shBC = jnp.dot(shv, BC_ref[...], preferred_element_type=jnp.float32)
        shBH = jnp.dot(shv, BH_ref[...], preferred_element_type=jnp.float32)
        outC_ref[...] = mC * gC_ref[...] * shBC
        outH_ref[...] = mH * gH_ref[...] * shBH

    return pl.pallas_call(
        body,
        grid=(N // be,),
        in_specs=[_blk(be, 1), _blk(be, 9), _blk(be, DC), _blk(be, DH)]
        + [_full(w.shape) for w in wsC] + [_full(w.shape) for w in wsH]
        + [_full(BC.shape), _full(BH.shape)],
        out_specs=[_blk(be, DC), _blk(be, DH)],
        out_shape=[jax.ShapeDtypeStruct((N, DC), jnp.float32),
                   jax.ShapeDtypeStruct((N, DH), jnp.float32)],
    )(emb, sh, gC, gH, *wsC, *wsH, BC, BH)


def _tc_bond(emb_b, sh_b, ga, gb, ws_bond, ws_x, Ax, Bx, blk0, nblk, be):
    Dx = Ax.shape[1]
    nb, nx = len(ws_bond), len(ws_x)

    def off(d):
        return pl.BlockSpec((be, d), lambda i: (blk0 + i, 0))

    def body(*refs):
        emb_ref, sh_ref, ga_ref, gb_ref = refs[:4]
        i = 4
        wb = refs[i:i + nb]; i += nb
        wx = refs[i:i + nx]; i += nx
        A_ref, B_ref, out_ref = refs[i:i + 3]
        ev = emb_ref[...]
        hf = _mlp_refs(ev, wb) * ga_ref[...] * gb_ref[...]
        mx = _mlp_refs(ev, wx)
        hA = jnp.dot(hf, A_ref[...], preferred_element_type=jnp.float32)
        sB = jnp.dot(sh_ref[...], B_ref[...], preferred_element_type=jnp.float32)
        out_ref[...] = mx * hA * sB

    return pl.pallas_call(
        body,
        grid=(nblk,),
        in_specs=[off(1), off(9), off(ga.shape[1]), off(gb.shape[1])]
        + [_full(w.shape) for w in ws_bond] + [_full(w.shape) for w in ws_x]
        + [_full(Ax.shape), _full(Bx.shape)],
        out_specs=_blk(be, Dx),
        out_shape=jax.ShapeDtypeStruct((nblk * be, Dx), jnp.float32),
    )(emb_b, sh_b, ga, gb, *ws_bond, *ws_x, Ax, Bx)


def kernel(sh, emb, f_in, edge_src, edge_dst, num_nodes, num_neighbors,
           HH_ind, CC_ind, CH_ind, fc1, fc2, fc_bond, fcHH, fcCC, fcCH, fcC, fcH,
           A1, B1, A2, B2, Ab, Bb, AHH, BHH, ACC, BCC, ACH, BCH, AC, BC, AH, BH):
    S = f_in.shape[0]
    E = sh.shape[0]
    NB = HH_ind.shape[0]

    inv = 1.0 / jnp.sqrt(jnp.asarray(num_neighbors, jnp.float32))
    w1l = _pad2(fc1[-1] * inv, 16, 128)
    w2l = _pad2(fc2[-1] * inv, 16, 256)
    wCl = _pad2(fcC[-1] * inv, 16, 512)
    wHl = _pad2(fcH[-1] * inv, 16, 128)

    srcf = edge_src.astype(jnp.float32).reshape(E, 1)
    dstf = edge_dst.astype(jnp.float32).reshape(E, 1)
    meta = jnp.pad(jnp.concatenate([srcf, dstf, emb, sh], axis=1),
                   ((0, 0), (0, 116)))
    inds = jnp.concatenate(
        [HH_ind, CC_ind, CH_ind, jnp.zeros((E - 3 * NB,), jnp.int32)])

    (p1,) = _tc_node_mm(f_in, [_pad2(A1, 2, 128)], 2000)
    g1, metag = _sc_gather_multi([(p1, edge_src), (meta, inds)])
    ef1 = _tc_edge(emb, sh, g1, list(fc1[:-1]) + [w1l], _pad2(B1, 9, 128), 1600)
    (nf1,) = _sc_scatter_multi([ef1], edge_dst, S)

    (p2,) = _tc_node_mm(nf1, [_pad2(A2, 128, 256)], 2000)
    (g2,) = _sc_gather_multi([(p2, edge_src)])
    ef2 = _tc_edge(emb, sh, g2, list(fc2[:-1]) + [w2l], _pad2(B2, 9, 256), 1600)
    (nf2,) = _sc_scatter_multi([ef2], edge_dst, S)

    TA, TB, TCt, THt = _tc_node_mm(
        nf2,
        [_pad2(Ab, 256, 256), _pad2(Bb, 256, 256),
         _pad2(AC, 256, 512), _pad2(AH, 256, 128)], 2000)

    gC, gH = _sc_gather_multi([(TCt, edge_src), (THt, edge_src)], K=64)
    edge_C, edge_H = _tc_edge_ch(
        emb, sh, gC, gH,
        list(fcC[:-1]) + [wCl], list(fcH[:-1]) + [wHl],
        _pad2(BC, 9, 512), _pad2(BH, 9, 128), 1600)
    node_C, node_H = _sc_scatter_multi([edge_C, edge_H], edge_dst, S)

    NBP = 122880
    b_src = metag[:NBP, 0].astype(jnp.int32)
    b_dst = metag[:NBP, 1].astype(jnp.int32)
    emb_b = metag[:NBP, 2:3]
    sh_b = metag[:NBP, 3:12]
    gA, gB = _sc_gather_multi([(TA, b_src), (TB, b_dst)], K=64)

    be = 2000
    nblk = NB // be
    wsb = list(fc_bond[:-1]) + [_pad2(fc_bond[-1], 16, 256)]
    edge_HH = _tc_bond(emb_b, sh_b, gA, gB, wsb, fcHH,
                       _pad2(AHH, 256, 50), BHH, 0 * nblk, nblk, be)
    edge_CC = _tc_bond(emb_b, sh_b, gA, gB, wsb, fcCC,
                       _pad2(ACC, 256, 392), BCC, 1 * nblk, nblk, be)
    edge_CH = _tc_bond(emb_b, sh_b, gA, gB, wsb, fcCH,
                       _pad2(ACH, 256, 140), BCH, 2 * nblk, nblk, be)

    hC, hH = 196, 25
    return (node_H[:, :hH], node_C[:, :hC], edge_HH[:, :hH], edge_CH[:, :70],
            edge_CC[:, :hC],
            node_H[:, hH:2 * hH], node_C[:, hC:2 * hC], edge_HH[:, hH:2 * hH],
            edge_CH[:, 70:140], edge_CC[:, hC:2 * hC])


# speedup vs baseline: 1.8518x; 1.5369x over previous
import functools

import jax
import jax.numpy as jnp
from jax import lax
from jax.experimental import pallas as pl
from jax.experimental.pallas import tpu as pltpu
from jax.experimental.pallas import tpu_sc as plsc

NC = 2
NS = 16
NW = NC * NS
K = 128


def _pad2(m, rows, cols):
    return jnp.pad(m, ((0, rows - m.shape[0]), (0, cols - m.shape[1])))


def _sc_gather_multi(pairs, K=128, nbuf=2):
    N = pairs[0][1].shape[0]
    n_chunks = N // K
    iters = (n_chunks + NW - 1) // NW
    rounds = (iters + nbuf - 1) // nbuf
    n_t = len(pairs)
    uidx, uslot = [], []
    for _, ix in pairs:
        for u, ux in enumerate(uidx):
            if ux is ix:
                uslot.append(u)
                break
        else:
            uslot.append(len(uidx))
            uidx.append(ix)
    n_u = len(uidx)
    mesh = plsc.VectorSubcoreMesh(core_axis_name="c", subcore_axis_name="s")

    scratch = (
        [pltpu.VMEM((K,), jnp.int32) for _ in range(n_u * nbuf)]
        + [pltpu.VMEM((K, t.shape[1]), jnp.float32)
           for t, _ in pairs for _ in range(nbuf)]
        + [pltpu.SemaphoreType.DMA for _ in range(2 * nbuf)]
    )

    @functools.partial(
        pl.kernel,
        mesh=mesh,
        out_type=[jax.ShapeDtypeStruct((N, t.shape[1]), jnp.float32)
                  for t, _ in pairs],
        scratch_types=scratch,
    )
    def k(*refs):
        tabs = refs[0:n_t]
        idxs = refs[n_t:n_t + n_u]
        outs = refs[n_t + n_u:n_t + n_u + n_t]
        sc = refs[n_t + n_u + n_t:]
        ib = [sc[u * nbuf:(u + 1) * nbuf] for u in range(n_u)]
        sc = sc[n_u * nbuf:]
        rb = [sc[j * nbuf:(j + 1) * nbuf] for j in range(n_t)]
        sc = sc[n_t * nbuf:]
        gsem, ssem = sc[:nbuf], sc[nbuf:2 * nbuf]
        wid = lax.axis_index("s") * NC + lax.axis_index("c")

        def gath(j, b):
            return pltpu.make_async_copy(
                tabs[j].at[ib[uslot[j]][b]], rb[j][b], gsem[b])

        def stor(j, b, off):
            return pltpu.make_async_copy(
                rb[j][b], outs[j].at[pl.ds(off, K)], ssem[b])

        def body(r, carry):
            cs = []
            for b in range(nbuf):
                cid = (r * nbuf + b) * NW + wid
                cs.append((cid < n_chunks, cid * K))
            for b in range(nbuf):
                pred, off = cs[b]

                @pl.when(pred)
                def _(b=b, off=off):
                    for u in range(n_u):
                        pltpu.sync_copy(idxs[u].at[pl.ds(off, K)], ib[u][b])
                    for j in range(n_t):
                        gath(j, b).start()

            for b in range(nbuf):
                pred, off = cs[b]

                @pl.when(pred)
                def _(b=b, off=off):
                    for j in range(n_t):
                        gath(j, b).wait()
                    for j in range(n_t):
                        stor(j, b, off).start()

            for b in range(nbuf):
                pred, off = cs[b]

                @pl.when(pred)
                def _(b=b, off=off):
                    for j in range(n_t):
                        stor(j, b, off).wait()

            return carry

        lax.fori_loop(0, rounds, body, 0)

    res = k(*[t for t, _ in pairs], *uidx)
    return list(res) if isinstance(res, (list, tuple)) else [res]


def _sc_scatter_multi(vals_list, idx, S):
    N = idx.shape[0]
    n_chunks = N // K
    iters = (n_chunks + NS - 1) // NS
    n_v = len(vals_list)
    zeros = jnp.zeros((S, 128), jnp.float32)
    mesh = plsc.VectorSubcoreMesh(core_axis_name="c", subcore_axis_name="s")
    phases = []
    for v, a in enumerate(vals_list):
        for cb in range(0, a.shape[1], 256):
            phases.append((v, cb))
    r_lo, r_hi = (S // 16) // 8 * 8, S - 15 * ((S // 16) // 8 * 8)

    nbuf = 3
    rounds = (iters + nbuf - 1) // nbuf

    @functools.partial(
        pl.kernel,
        mesh=mesh,
        out_type=[jax.ShapeDtypeStruct((S, a.shape[1]), jnp.float32)
                  for a in vals_list],
        scratch_types=(
            [pltpu.VMEM((K,), jnp.int32) for _ in range(nbuf)]
            + [pltpu.VMEM((K, 128), jnp.float32) for _ in range(nbuf)]
            + [pltpu.VMEM_SHARED((S, 128), jnp.float32)]
            + [pltpu.SemaphoreType.DMA for _ in range(2 * nbuf)]
        ),
    )
    def k(*refs):
        vals = refs[0:n_v]
        idx_hbm = refs[n_v]
        zeros_hbm = refs[n_v + 1]
        outs = refs[n_v + 2:n_v + 2 + n_v]
        sc = refs[n_v + 2 + n_v:]
        ib, vb = sc[:nbuf], sc[nbuf:2 * nbuf]
        acc = sc[2 * nbuf]
        lsem = sc[2 * nbuf + 1:2 * nbuf + 1 + nbuf]
        asem = sc[2 * nbuf + 1 + nbuf:2 * nbuf + 1 + 2 * nbuf]
        c = lax.axis_index("c")
        s = lax.axis_index("s")
        r0 = s * r_lo

        for v, cb in phases:
            D = vals_list[v].shape[1]
            col = cb + c * 128
            active = col < D

            @pl.when(active & (s < 15))
            def _():
                pltpu.sync_copy(zeros_hbm.at[pl.ds(r0, r_lo)],
                                acc.at[pl.ds(r0, r_lo)])

            @pl.when(active & (s == 15))
            def _():
                pltpu.sync_copy(zeros_hbm.at[pl.ds(15 * r_lo, r_hi)],
                                acc.at[pl.ds(15 * r_lo, r_hi)])

            plsc.subcore_barrier()

            def ldi(b, off):
                return pltpu.make_async_copy(
                    idx_hbm.at[pl.ds(off, K)], ib[b], lsem[b])

            def ldv(b, off):
                return pltpu.make_async_copy(
                    vals[v].at[pl.ds(off, K), pl.ds(col, 128)], vb[b], lsem[b])

            def addv(b):
                return pltpu.make_async_copy(vb[b], acc.at[ib[b]], asem[b])

            def body(i, carry):
                cs = []
                for b in range(nbuf):
                    cid = (i * nbuf + b) * NS + s
                    cs.append((active & (cid < n_chunks), cid * K))
                for b in range(nbuf):
                    pred, off = cs[b]

                    @pl.when(pred)
                    def _(b=b, off=off):
                        ldi(b, off).start()
                        ldv(b, off).start()

                for b in range(nbuf):
                    pred, off = cs[b]

                    @pl.when(pred)
                    def _(b=b, off=off):
                        ldi(b, off).wait()
                        ldv(b, off).wait()
                        addv(b).start(add=True)

                for b in range(nbuf):
                    pred, off = cs[b]

                    @pl.when(pred)
                    def _(b=b):
                        addv(b).wait()

                return carry

            lax.fori_loop(0, rounds, body, 0)
            plsc.subcore_barrier()

            @pl.when(active & (s < 15))
            def _():
                pltpu.sync_copy(acc.at[pl.ds(r0, r_lo)],
                                outs[v].at[pl.ds(r0, r_lo), pl.ds(col, 128)])

            @pl.when(active & (s == 15))
            def _():
                pltpu.sync_copy(acc.at[pl.ds(15 * r_lo, r_hi)],
                                outs[v].at[pl.ds(15 * r_lo, r_hi), pl.ds(col, 128)])

            plsc.subcore_barrier()

    res = k(*vals_list, idx, zeros)
    return list(res) if isinstance(res, (list, tuple)) else [res]


def _mlp_refs(x, wrefs):
    h = x * wrefs[0][...]
    for w in wrefs[1:]:
        h = jnp.maximum(h, 0.0)
        h = jnp.dot(h, w[...], preferred_element_type=jnp.float32)
    return h


def _full(shape):
    return pl.BlockSpec(shape, lambda i: tuple(0 for _ in shape))


def _blk(be, d):
    return pl.BlockSpec((be, d), lambda i: (i, 0))


def _tc_edge(emb, sh, g, ws, Bm, be):
    N, D = g.shape
    nw = len(ws)

    def body(*refs):
        emb_ref, sh_ref, g_ref = refs[:3]
        wrefs = refs[3:3 + nw]
        B_ref = refs[3 + nw]
        out_ref = refs[3 + nw + 1]
        w = _mlp_refs(emb_ref[...], wrefs)
        shB = jnp.dot(sh_ref[...], B_ref[...], preferred_element_type=jnp.float32)
        out_ref[...] = w * g_ref[...] * shB

    return pl.pallas_call(
        body,
        grid=(N // be,),
        in_specs=[_blk(be, 1), _blk(be, 9), _blk(be, D)]
        + [_full(w.shape) for w in ws] + [_full(Bm.shape)],
        out_specs=_blk(be, D),
        out_shape=jax.ShapeDtypeStruct((N, D), jnp.float32),
    )(emb, sh, g, *ws, Bm)


def _tc_node_mm(x, mats, br):
    S, Din = x.shape

    def body(*refs):
        x_ref = refs[0]
        m_refs = refs[1:1 + len(mats)]
        out_refs = refs[1 + len(mats):]
        xv = x_ref[...]
        for m_ref, o_ref in zip(m_refs, out_refs):
            o_ref[...] = jnp.dot(xv, m_ref[...], preferred_element_type=jnp.float32)

    return pl.pallas_call(
        body,
        grid=(S // br,),
        in_specs=[_blk(br, Din)] + [_full(m.shape) for m in mats],
        out_specs=[_blk(br, m.shape[1]) for m in mats],
        out_shape=[jax.ShapeDtypeStruct((S, m.shape[1]), jnp.float32) for m in mats],
    )(x, *mats)


def _tc_edge_ch(emb, sh, gC, gH, wsC, wsH, BC, BH, be):
    N = gC.shape[0]
    DC, DH = gC.shape[1], gH.shape[1]

    def body(*refs):
        emb_ref, sh_ref, gC_ref, gH_ref = refs[:4]
        i = 4
        wC = refs[i:i + 4]; i += 4
        wH = refs[i:i + 4]; i += 4
        BC_ref, BH_ref, outC_ref, outH_ref = refs[i:i + 4]
        shv = sh_ref[...]
        ev = emb_ref[...]
        mC = _mlp_refs(ev, wC)
        mH = _mlp_refs(ev, wH)
        shBC = jnp.dot(shv, BC_ref[...], preferred_element_type=jnp.float32)
        shBH = jnp.dot(shv, BH_ref[...], preferred_element_type=jnp.float32)
        outC_ref[...] = mC * gC_ref[...] * shBC
        outH_ref[...] = mH * gH_ref[...] * shBH

    return pl.pallas_call(
        body,
        grid=(N // be,),
        in_specs=[_blk(be, 1), _blk(be, 9), _blk(be, DC), _blk(be, DH)]
        + [_full(w.shape) for w in wsC] + [_full(w.shape) for w in wsH]
        + [_full(BC.shape), _full(BH.shape)],
        out_specs=[_blk(be, DC), _blk(be, DH)],
        out_shape=[jax.ShapeDtypeStruct((N, DC), jnp.float32),
                   jax.ShapeDtypeStruct((N, DH), jnp.float32)],
    )(emb, sh, gC, gH, *wsC, *wsH, BC, BH)


def _tc_bond(emb_b, sh_b, ga, gb, ws_bond, ws_x, Ax, Bx, blk0, nblk, be):
    Dx = Ax.shape[1]
    nb, nx = len(ws_bond), len(ws_x)

    def off(d):
        return pl.BlockSpec((be, d), lambda i: (blk0 + i, 0))

    def body(*refs):
        emb_ref, sh_ref, ga_ref, gb_ref = refs[:4]
        i = 4
        wb = refs[i:i + nb]; i += nb
        wx = refs[i:i + nx]; i += nx
        A_ref, B_ref, out_ref = refs[i:i + 3]
        ev = emb_ref[...]
        hf = _mlp_refs(ev, wb) * ga_ref[...] * gb_ref[...]
        mx = _mlp_refs(ev, wx)
        hA = jnp.dot(hf, A_ref[...], preferred_element_type=jnp.float32)
        sB = jnp.dot(sh_ref[...], B_ref[...], preferred_element_type=jnp.float32)
        out_ref[...] = mx * hA * sB

    return pl.pallas_call(
        body,
        grid=(nblk,),
        in_specs=[off(1), off(9), off(ga.shape[1]), off(gb.shape[1])]
        + [_full(w.shape) for w in ws_bond] + [_full(w.shape) for w in ws_x]
        + [_full(Ax.shape), _full(Bx.shape)],
        out_specs=_blk(be, Dx),
        out_shape=jax.ShapeDtypeStruct((nblk * be, Dx), jnp.float32),
    )(emb_b, sh_b, ga, gb, *ws_bond, *ws_x, Ax, Bx)


def kernel(sh, emb, f_in, edge_src, edge_dst, num_nodes, num_neighbors,
           HH_ind, CC_ind, CH_ind, fc1, fc2, fc_bond, fcHH, fcCC, fcCH, fcC, fcH,
           A1, B1, A2, B2, Ab, Bb, AHH, BHH, ACC, BCC, ACH, BCH, AC, BC, AH, BH):
    S = f_in.shape[0]
    E = sh.shape[0]
    NB = HH_ind.shape[0]
    NBP = 122880

    inv = 1.0 / jnp.sqrt(jnp.asarray(num_neighbors, jnp.float32))
    w1l = _pad2(fc1[-1] * inv, 16, 128)
    w2l = _pad2(fc2[-1] * inv, 16, 256)
    wCl = _pad2(fcC[-1] * inv, 16, 512)
    wHl = _pad2(fcH[-1] * inv, 16, 128)

    srcf = edge_src.astype(jnp.float32).reshape(E, 1)
    dstf = edge_dst.astype(jnp.float32).reshape(E, 1)
    meta = jnp.pad(jnp.concatenate([srcf, dstf, emb, sh], axis=1),
                   ((0, 0), (0, 116)))
    inds = jnp.concatenate(
        [HH_ind, CC_ind, CH_ind, jnp.zeros((E - 3 * NB,), jnp.int32)])

    (p1,) = _tc_node_mm(f_in, [_pad2(A1, 2, 128)], 2000)
    (g1,) = _sc_gather_multi([(p1, edge_src)], nbuf=3)
    ef1 = _tc_edge(emb, sh, g1, list(fc1[:-1]) + [w1l], _pad2(B1, 9, 128), 1600)
    (nf1,) = _sc_scatter_multi([ef1], edge_dst, S)

    (p2,) = _tc_node_mm(nf1, [_pad2(A2, 128, 256)], 2000)
    (g2,) = _sc_gather_multi([(p2, edge_src)], nbuf=3)
    ef2 = _tc_edge(emb, sh, g2, list(fc2[:-1]) + [w2l], _pad2(B2, 9, 256), 1600)
    (nf2,) = _sc_scatter_multi([ef2], edge_dst, S)

    TA, TB, TCt, THt = _tc_node_mm(
        nf2,
        [_pad2(Ab, 256, 256), _pad2(Bb, 256, 256),
         _pad2(AC, 256, 512), _pad2(AH, 256, 128)], 2000)

    gC, gH = _sc_gather_multi([(TCt, edge_src), (THt, edge_src)], K=64)
    (metag,) = _sc_gather_multi([(meta, inds[:NBP])], nbuf=4)
    edge_C, edge_H = _tc_edge_ch(
        emb, sh, gC, gH,
        list(fcC[:-1]) + [wCl], list(fcH[:-1]) + [wHl],
        _pad2(BC, 9, 512), _pad2(BH, 9, 128), 1600)
    node_C, node_H = _sc_scatter_multi([edge_C, edge_H], edge_dst, S)

    b_src = metag[:NBP, 0].astype(jnp.int32)
    b_dst = metag[:NBP, 1].astype(jnp.int32)
    emb_b = metag[:NBP, 2:3]
    sh_b = metag[:NBP, 3:12]
    gA, gB = _sc_gather_multi([(TA, b_src), (TB, b_dst)], K=96)

    be = 2000
    nblk = NB // be
    wsb = list(fc_bond[:-1]) + [_pad2(fc_bond[-1], 16, 256)]
    edge_HH = _tc_bond(emb_b, sh_b, gA, gB, wsb, fcHH,
                       _pad2(AHH, 256, 50), BHH, 0 * nblk, nblk, be)
    edge_CC = _tc_bond(emb_b, sh_b, gA, gB, wsb, fcCC,
                       _pad2(ACC, 256, 392), BCC, 1 * nblk, nblk, be)
    edge_CH = _tc_bond(emb_b, sh_b, gA, gB, wsb, fcCH,
                       _pad2(ACH, 256, 140), BCH, 2 * nblk, nblk, be)

    hC, hH = 196, 25
    return (node_H[:, :hH], node_C[:, :hC], edge_HH[:, :hH], edge_CH[:, :70],
            edge_CC[:, :hC],
            node_H[:, hH:2 * hH], node_C[:, hC:2 * hC], edge_HH[:, hH:2 * hH],
            edge_CH[:, 70:140], edge_CC[:, hC:2 * hC])


# speedup vs baseline: 1.9496x; 1.0528x over previous
import functools

import jax
import jax.numpy as jnp
from jax import lax
from jax.experimental import pallas as pl
from jax.experimental.pallas import tpu as pltpu
from jax.experimental.pallas import tpu_sc as plsc

NC = 2
NS = 16
NW = NC * NS
K = 128


def _pad2(m, rows, cols):
    return jnp.pad(m, ((0, rows - m.shape[0]), (0, cols - m.shape[1])))


def _sc_gather_multi(pairs, K=128, nbuf=2):
    N = pairs[0][1].shape[0]
    n_chunks = N // K
    iters = (n_chunks + NW - 1) // NW
    rounds = (iters + nbuf - 1) // nbuf
    n_t = len(pairs)
    uidx, uslot = [], []
    for _, ix in pairs:
        for u, ux in enumerate(uidx):
            if ux is ix:
                uslot.append(u)
                break
        else:
            uslot.append(len(uidx))
            uidx.append(ix)
    n_u = len(uidx)
    mesh = plsc.VectorSubcoreMesh(core_axis_name="c", subcore_axis_name="s")

    scratch = (
        [pltpu.VMEM((K,), jnp.int32) for _ in range(n_u * nbuf)]
        + [pltpu.VMEM((K, t.shape[1]), jnp.float32)
           for t, _ in pairs for _ in range(nbuf)]
        + [pltpu.SemaphoreType.DMA for _ in range(2 * nbuf)]
    )

    @functools.partial(
        pl.kernel,
        mesh=mesh,
        out_type=[jax.ShapeDtypeStruct((N, t.shape[1]), jnp.float32)
                  for t, _ in pairs],
        scratch_types=scratch,
    )
    def k(*refs):
        tabs = refs[0:n_t]
        idxs = refs[n_t:n_t + n_u]
        outs = refs[n_t + n_u:n_t + n_u + n_t]
        sc = refs[n_t + n_u + n_t:]
        ib = [sc[u * nbuf:(u + 1) * nbuf] for u in range(n_u)]
        sc = sc[n_u * nbuf:]
        rb = [sc[j * nbuf:(j + 1) * nbuf] for j in range(n_t)]
        sc = sc[n_t * nbuf:]
        gsem, ssem = sc[:nbuf], sc[nbuf:2 * nbuf]
        wid = lax.axis_index("s") * NC + lax.axis_index("c")

        def gath(j, b):
            return pltpu.make_async_copy(
                tabs[j].at[ib[uslot[j]][b]], rb[j][b], gsem[b])

        def stor(j, b, off):
            return pltpu.make_async_copy(
                rb[j][b], outs[j].at[pl.ds(off, K)], ssem[b])

        def body(r, carry):
            cs = []
            for b in range(nbuf):
                cid = (r * nbuf + b) * NW + wid
                cs.append((cid < n_chunks, cid * K))
            for b in range(nbuf):
                pred, off = cs[b]

                @pl.when(pred)
                def _(b=b, off=off):
                    for u in range(n_u):
                        pltpu.sync_copy(idxs[u].at[pl.ds(off, K)], ib[u][b])
                    for j in range(n_t):
                        gath(j, b).start()

            for b in range(nbuf):
                pred, off = cs[b]

                @pl.when(pred)
                def _(b=b, off=off):
                    for j in range(n_t):
                        gath(j, b).wait()
                    for j in range(n_t):
                        stor(j, b, off).start()

            for b in range(nbuf):
                pred, off = cs[b]

                @pl.when(pred)
                def _(b=b, off=off):
                    for j in range(n_t):
                        stor(j, b, off).wait()

            return carry

        lax.fori_loop(0, rounds, body, 0)

    res = k(*[t for t, _ in pairs], *uidx)
    return list(res) if isinstance(res, (list, tuple)) else [res]


def _sc_scatter_multi(vals_list, idx, S):
    N = idx.shape[0]
    n_chunks = N // K
    iters = (n_chunks + NS - 1) // NS
    n_v = len(vals_list)
    zeros = jnp.zeros((S, 128), jnp.float32)
    mesh = plsc.VectorSubcoreMesh(core_axis_name="c", subcore_axis_name="s")
    phases = []
    for v, a in enumerate(vals_list):
        for cb in range(0, a.shape[1], 256):
            phases.append((v, cb))
    r_lo, r_hi = (S // 16) // 8 * 8, S - 15 * ((S // 16) // 8 * 8)

    nbuf = 3
    rounds = (iters + nbuf - 1) // nbuf

    @functools.partial(
        pl.kernel,
        mesh=mesh,
        out_type=[jax.ShapeDtypeStruct((S, a.shape[1]), jnp.float32)
                  for a in vals_list],
        scratch_types=(
            [pltpu.VMEM((K,), jnp.int32) for _ in range(nbuf)]
            + [pltpu.VMEM((K, 128), jnp.float32) for _ in range(nbuf)]
            + [pltpu.VMEM_SHARED((S, 128), jnp.float32)]
            + [pltpu.SemaphoreType.DMA for _ in range(2 * nbuf)]
        ),
    )
    def k(*refs):
        vals = refs[0:n_v]
        idx_hbm = refs[n_v]
        zeros_hbm = refs[n_v + 1]
        outs = refs[n_v + 2:n_v + 2 + n_v]
        sc = refs[n_v + 2 + n_v:]
        ib, vb = sc[:nbuf], sc[nbuf:2 * nbuf]
        acc = sc[2 * nbuf]
        lsem = sc[2 * nbuf + 1:2 * nbuf + 1 + nbuf]
        asem = sc[2 * nbuf + 1 + nbuf:2 * nbuf + 1 + 2 * nbuf]
        c = lax.axis_index("c")
        s = lax.axis_index("s")
        r0 = s * r_lo

        for v, cb in phases:
            D = vals_list[v].shape[1]
            col = cb + c * 128
            active = col < D

            @pl.when(active & (s < 15))
            def _():
                pltpu.sync_copy(zeros_hbm.at[pl.ds(r0, r_lo)],
                                acc.at[pl.ds(r0, r_lo)])

            @pl.when(active & (s == 15))
            def _():
                pltpu.sync_copy(zeros_hbm.at[pl.ds(15 * r_lo, r_hi)],
                                acc.at[pl.ds(15 * r_lo, r_hi)])

            plsc.subcore_barrier()

            def ldi(b, off):
                return pltpu.make_async_copy(
                    idx_hbm.at[pl.ds(off, K)], ib[b], lsem[b])

            def ldv(b, off):
                return pltpu.make_async_copy(
                    vals[v].at[pl.ds(off, K), pl.ds(col, 128)], vb[b], lsem[b])

            def addv(b):
                return pltpu.make_async_copy(vb[b], acc.at[ib[b]], asem[b])

            def body(i, carry):
                cs = []
                for b in range(nbuf):
                    cid = (i * nbuf + b) * NS + s
                    cs.append((active & (cid < n_chunks), cid * K))
                for b in range(nbuf):
                    pred, off = cs[b]

                    @pl.when(pred)
                    def _(b=b, off=off):
                        ldi(b, off).start()
                        ldv(b, off).start()

                for b in range(nbuf):
                    pred, off = cs[b]

                    @pl.when(pred)
                    def _(b=b, off=off):
                        ldi(b, off).wait()
                        ldv(b, off).wait()
                        addv(b).start(add=True)

                for b in range(nbuf):
                    pred, off = cs[b]

                    @pl.when(pred)
                    def _(b=b):
                        addv(b).wait()

                return carry

            lax.fori_loop(0, rounds, body, 0)
            plsc.subcore_barrier()

            @pl.when(active & (s < 15))
            def _():
                pltpu.sync_copy(acc.at[pl.ds(r0, r_lo)],
                                outs[v].at[pl.ds(r0, r_lo), pl.ds(col, 128)])

            @pl.when(active & (s == 15))
            def _():
                pltpu.sync_copy(acc.at[pl.ds(15 * r_lo, r_hi)],
                                outs[v].at[pl.ds(15 * r_lo, r_hi), pl.ds(col, 128)])

            plsc.subcore_barrier()

    res = k(*vals_list, idx, zeros)
    return list(res) if isinstance(res, (list, tuple)) else [res]


def _mlp_refs(x, wrefs):
    h = x * wrefs[0][...]
    for w in wrefs[1:]:
        h = jnp.maximum(h, 0.0)
        h = jnp.dot(h, w[...], preferred_element_type=jnp.float32)
    return h


def _full(shape):
    return pl.BlockSpec(shape, lambda i: tuple(0 for _ in shape))


def _blk(be, d):
    return pl.BlockSpec((be, d), lambda i: (i, 0))


def _tc_edge(emb, sh, g, ws, Bm, be):
    N, D = g.shape
    nw = len(ws)

    def body(*refs):
        emb_ref, sh_ref, g_ref = refs[:3]
        wrefs = refs[3:3 + nw]
        B_ref = refs[3 + nw]
        out_ref = refs[3 + nw + 1]
        w = _mlp_refs(emb_ref[...], wrefs)
        shB = jnp.dot(sh_ref[...], B_ref[...], preferred_element_type=jnp.float32)
        out_ref[...] = w * g_ref[...] * shB

    return pl.pallas_call(
        body,
        grid=(N // be,),
        in_specs=[_blk(be, 1), _blk(be, 9), _blk(be, D)]
        + [_full(w.shape) for w in ws] + [_full(Bm.shape)],
        out_specs=_blk(be, D),
        out_shape=jax.ShapeDtypeStruct((N, D), jnp.float32),
    )(emb, sh, g, *ws, Bm)


def _tc_node_mm(x, mats, br):
    S, Din = x.shape

    def body(*refs):
        x_ref = refs[0]
        m_refs = refs[1:1 + len(mats)]
        out_refs = refs[1 + len(mats):]
        xv = x_ref[...]
        for m_ref, o_ref in zip(m_refs, out_refs):
            o_ref[...] = jnp.dot(xv, m_ref[...], preferred_element_type=jnp.float32)

    return pl.pallas_call(
        body,
        grid=(S // br,),
        in_specs=[_blk(br, Din)] + [_full(m.shape) for m in mats],
        out_specs=[_blk(br, m.shape[1]) for m in mats],
        out_shape=[jax.ShapeDtypeStruct((S, m.shape[1]), jnp.float32) for m in mats],
    )(x, *mats)


def _tc_edge_ch(emb, sh, g, wsC, wsH, BC, BH, be):
    N, D = g.shape

    def body(*refs):
        emb_ref, sh_ref, g_ref = refs[:3]
        i = 3
        wC = refs[i:i + 4]; i += 4
        wH = refs[i:i + 4]; i += 4
        BC_ref, BH_ref, out_ref = refs[i:i + 3]
        shv = sh_ref[...]
        ev = emb_ref[...]
        mC = _mlp_refs(ev, wC)
        mH = _mlp_refs(ev, wH)
        shBC = jnp.dot(shv, BC_ref[...], preferred_element_type=jnp.float32)
        shBH = jnp.dot(shv, BH_ref[...], preferred_element_type=jnp.float32)
        out_ref[...] = g_ref[...] * (mC * shBC + mH * shBH)

    return pl.pallas_call(
        body,
        grid=(N // be,),
        in_specs=[_blk(be, 1), _blk(be, 9), _blk(be, D)]
        + [_full(w.shape) for w in wsC] + [_full(w.shape) for w in wsH]
        + [_full(BC.shape), _full(BH.shape)],
        out_specs=_blk(be, D),
        out_shape=jax.ShapeDtypeStruct((N, D), jnp.float32),
    )(emb, sh, g, *wsC, *wsH, BC, BH)


def _tc_bond(emb_b, sh_b, ga, gb, ws_bond, ws_x, Ax, Bx, blk0, nblk, be):
    Dx = Ax.shape[1]
    nb, nx = len(ws_bond), len(ws_x)

    def off(d):
        return pl.BlockSpec((be, d), lambda i: (blk0 + i, 0))

    def body(*refs):
        emb_ref, sh_ref, ga_ref, gb_ref = refs[:4]
        i = 4
        wb = refs[i:i + nb]; i += nb
        wx = refs[i:i + nx]; i += nx
        A_ref, B_ref, out_ref = refs[i:i + 3]
        ev = emb_ref[...]
        hf = _mlp_refs(ev, wb) * ga_ref[...] * gb_ref[...]
        mx = _mlp_refs(ev, wx)
        hA = jnp.dot(hf, A_ref[...], preferred_element_type=jnp.float32)
        sB = jnp.dot(sh_ref[...], B_ref[...], preferred_element_type=jnp.float32)
        out_ref[...] = mx * hA * sB

    return pl.pallas_call(
        body,
        grid=(nblk,),
        in_specs=[off(1), off(9), off(ga.shape[1]), off(gb.shape[1])]
        + [_full(w.shape) for w in ws_bond] + [_full(w.shape) for w in ws_x]
        + [_full(Ax.shape), _full(Bx.shape)],
        out_specs=_blk(be, Dx),
        out_shape=jax.ShapeDtypeStruct((nblk * be, Dx), jnp.float32),
    )(emb_b, sh_b, ga, gb, *ws_bond, *ws_x, Ax, Bx)


def kernel(sh, emb, f_in, edge_src, edge_dst, num_nodes, num_neighbors,
           HH_ind, CC_ind, CH_ind, fc1, fc2, fc_bond, fcHH, fcCC, fcCH, fcC, fcH,
           A1, B1, A2, B2, Ab, Bb, AHH, BHH, ACC, BCC, ACH, BCH, AC, BC, AH, BH):
    S = f_in.shape[0]
    E = sh.shape[0]
    NB = HH_ind.shape[0]
    NBP = 122880

    inv = 1.0 / jnp.sqrt(jnp.asarray(num_neighbors, jnp.float32))
    w1l = _pad2(fc1[-1] * inv, 16, 128)
    w2l = _pad2(fc2[-1] * inv, 16, 256)
    wCl = _pad2(fcC[-1] * inv, 16, 512)
    wHl = jnp.pad(fcH[-1] * inv, ((0, 0), (392, 70)))
    BHe = jnp.pad(BH, ((0, 0), (392, 70)))

    srcf = edge_src.astype(jnp.float32).reshape(E, 1)
    dstf = edge_dst.astype(jnp.float32).reshape(E, 1)
    meta = jnp.pad(jnp.concatenate([srcf, dstf, emb, sh], axis=1),
                   ((0, 0), (0, 116)))
    inds = jnp.concatenate(
        [HH_ind, CC_ind, CH_ind, jnp.zeros((E - 3 * NB,), jnp.int32)])

    (p1,) = _tc_node_mm(f_in, [_pad2(A1, 2, 128)], 2000)
    (g1,) = _sc_gather_multi([(p1, edge_src)], nbuf=3)
    ef1 = _tc_edge(emb, sh, g1, list(fc1[:-1]) + [w1l], _pad2(B1, 9, 128), 1600)
    (nf1,) = _sc_scatter_multi([ef1], edge_dst, S)

    (p2,) = _tc_node_mm(nf1, [_pad2(A2, 128, 256)], 2000)
    (g2,) = _sc_gather_multi([(p2, edge_src)], nbuf=3)
    ef2 = _tc_edge(emb, sh, g2, list(fc2[:-1]) + [w2l], _pad2(B2, 9, 256), 1600)
    (nf2,) = _sc_scatter_multi([ef2], edge_dst, S)

    ACHw = _pad2(jnp.concatenate([AC, AH], axis=1), 256, 512)
    TA, TB, TCHt = _tc_node_mm(
        nf2, [_pad2(Ab, 256, 256), _pad2(Bb, 256, 256), ACHw], 2000)

    (gCH,) = _sc_gather_multi([(TCHt, edge_src)], K=64, nbuf=3)
    (metag,) = _sc_gather_multi([(meta, inds[:NBP])], nbuf=4)
    edge_CH = _tc_edge_ch(
        emb, sh, gCH,
        list(fcC[:-1]) + [wCl], list(fcH[:-1]) + [wHl],
        _pad2(BC, 9, 512), BHe, 1600)
    (node_CH,) = _sc_scatter_multi([edge_CH], edge_dst, S)
    node_C = node_CH[:, :392]
    node_H = node_CH[:, 392:442]

    b_src = metag[:NBP, 0].astype(jnp.int32)
    b_dst = metag[:NBP, 1].astype(jnp.int32)
    emb_b = metag[:NBP, 2:3]
    sh_b = metag[:NBP, 3:12]
    gA, gB = _sc_gather_multi([(TA, b_src), (TB, b_dst)], K=64, nbuf=3)

    be = 2000
    nblk = NB // be
    wsb = list(fc_bond[:-1]) + [_pad2(fc_bond[-1], 16, 256)]
    edge_HH = _tc_bond(emb_b, sh_b, gA, gB, wsb, fcHH,
                       _pad2(AHH, 256, 50), BHH, 0 * nblk, nblk, be)
    edge_CC = _tc_bond(emb_b, sh_b, gA, gB, wsb, fcCC,
                       _pad2(ACC, 256, 392), BCC, 1 * nblk, nblk, be)
    edge_CH = _tc_bond(emb_b, sh_b, gA, gB, wsb, fcCH,
                       _pad2(ACH, 256, 140), BCH, 2 * nblk, nblk, be)

    hC, hH = 196, 25
    return (node_H[:, :hH], node_C[:, :hC], edge_HH[:, :hH], edge_CH[:, :70],
            edge_CC[:, :hC],
            node_H[:, hH:2 * hH], node_C[:, hC:2 * hC], edge_HH[:, hH:2 * hH],
            edge_CH[:, 70:140], edge_CC[:, hC:2 * hC])


# speedup vs baseline: 2.0531x; 1.0531x over previous
import functools

import jax
import jax.numpy as jnp
from jax import lax
from jax.experimental import pallas as pl
from jax.experimental.pallas import tpu as pltpu
from jax.experimental.pallas import tpu_sc as plsc

NC = 2
NS = 16
NW = NC * NS
K = 128


def _pad2(m, rows, cols):
    return jnp.pad(m, ((0, rows - m.shape[0]), (0, cols - m.shape[1])))


def _sc_gather_multi(pairs, K=128, nbuf=2):
    N = pairs[0][1].shape[0]
    n_chunks = N // K
    iters = (n_chunks + NW - 1) // NW
    rounds = (iters + nbuf - 1) // nbuf
    n_t = len(pairs)
    uidx, uslot = [], []
    for _, ix in pairs:
        for u, ux in enumerate(uidx):
            if ux is ix:
                uslot.append(u)
                break
        else:
            uslot.append(len(uidx))
            uidx.append(ix)
    n_u = len(uidx)
    mesh = plsc.VectorSubcoreMesh(core_axis_name="c", subcore_axis_name="s")

    scratch = (
        [pltpu.VMEM((K,), jnp.int32) for _ in range(n_u * nbuf)]
        + [pltpu.VMEM((K, t.shape[1]), jnp.float32)
           for t, _ in pairs for _ in range(nbuf)]
        + [pltpu.SemaphoreType.DMA for _ in range(2 * nbuf)]
    )

    @functools.partial(
        pl.kernel,
        mesh=mesh,
        out_type=[jax.ShapeDtypeStruct((N, t.shape[1]), jnp.float32)
                  for t, _ in pairs],
        scratch_types=scratch,
    )
    def k(*refs):
        tabs = refs[0:n_t]
        idxs = refs[n_t:n_t + n_u]
        outs = refs[n_t + n_u:n_t + n_u + n_t]
        sc = refs[n_t + n_u + n_t:]
        ib = [sc[u * nbuf:(u + 1) * nbuf] for u in range(n_u)]
        sc = sc[n_u * nbuf:]
        rb = [sc[j * nbuf:(j + 1) * nbuf] for j in range(n_t)]
        sc = sc[n_t * nbuf:]
        gsem, ssem = sc[:nbuf], sc[nbuf:2 * nbuf]
        wid = lax.axis_index("s") * NC + lax.axis_index("c")

        def gath(j, b):
            return pltpu.make_async_copy(
                tabs[j].at[ib[uslot[j]][b]], rb[j][b], gsem[b])

        def stor(j, b, off):
            return pltpu.make_async_copy(
                rb[j][b], outs[j].at[pl.ds(off, K)], ssem[b])

        def body(r, carry):
            cs = []
            for b in range(nbuf):
                cid = (r * nbuf + b) * NW + wid
                cs.append((cid < n_chunks, cid * K))
            for b in range(nbuf):
                pred, off = cs[b]

                @pl.when(pred)
                def _(b=b, off=off):
                    for u in range(n_u):
                        pltpu.sync_copy(idxs[u].at[pl.ds(off, K)], ib[u][b])
                    for j in range(n_t):
                        gath(j, b).start()

            for b in range(nbuf):
                pred, off = cs[b]

                @pl.when(pred)
                def _(b=b, off=off):
                    for j in range(n_t):
                        gath(j, b).wait()
                    for j in range(n_t):
                        stor(j, b, off).start()

            for b in range(nbuf):
                pred, off = cs[b]

                @pl.when(pred)
                def _(b=b, off=off):
                    for j in range(n_t):
                        stor(j, b, off).wait()

            return carry

        lax.fori_loop(0, rounds, body, 0)

    res = k(*[t for t, _ in pairs], *uidx)
    return list(res) if isinstance(res, (list, tuple)) else [res]


def _sc_scatter_multi(vals_list, idx, S):
    N = idx.shape[0]
    n_chunks = N // K
    iters = (n_chunks + NS - 1) // NS
    n_v = len(vals_list)
    zeros = jnp.zeros((S, 128), jnp.float32)
    mesh = plsc.VectorSubcoreMesh(core_axis_name="c", subcore_axis_name="s")
    phases = []
    for v, a in enumerate(vals_list):
        for cb in range(0, a.shape[1], 256):
            phases.append((v, cb))
    r_lo, r_hi = (S // 16) // 8 * 8, S - 15 * ((S // 16) // 8 * 8)

    nbuf = 3
    rounds = (iters + nbuf - 1) // nbuf

    @functools.partial(
        pl.kernel,
        mesh=mesh,
        out_type=[jax.ShapeDtypeStruct((S, a.shape[1]), jnp.float32)
                  for a in vals_list],
        scratch_types=(
            [pltpu.VMEM((K,), jnp.int32) for _ in range(nbuf)]
            + [pltpu.VMEM((K, 128), jnp.float32) for _ in range(nbuf)]
            + [pltpu.VMEM_SHARED((S, 128), jnp.float32)]
            + [pltpu.SemaphoreType.DMA for _ in range(2 * nbuf)]
        ),
    )
    def k(*refs):
        vals = refs[0:n_v]
        idx_hbm = refs[n_v]
        zeros_hbm = refs[n_v + 1]
        outs = refs[n_v + 2:n_v + 2 + n_v]
        sc = refs[n_v + 2 + n_v:]
        ib, vb = sc[:nbuf], sc[nbuf:2 * nbuf]
        acc = sc[2 * nbuf]
        lsem = sc[2 * nbuf + 1:2 * nbuf + 1 + nbuf]
        asem = sc[2 * nbuf + 1 + nbuf:2 * nbuf + 1 + 2 * nbuf]
        c = lax.axis_index("c")
        s = lax.axis_index("s")
        r0 = s * r_lo

        for v, cb in phases:
            D = vals_list[v].shape[1]
            col = cb + c * 128
            active = col < D

            @pl.when(active & (s < 15))
            def _():
                pltpu.sync_copy(zeros_hbm.at[pl.ds(r0, r_lo)],
                                acc.at[pl.ds(r0, r_lo)])

            @pl.when(active & (s == 15))
            def _():
                pltpu.sync_copy(zeros_hbm.at[pl.ds(15 * r_lo, r_hi)],
                                acc.at[pl.ds(15 * r_lo, r_hi)])

            plsc.subcore_barrier()

            def ldi(b, off):
                return pltpu.make_async_copy(
                    idx_hbm.at[pl.ds(off, K)], ib[b], lsem[b])

            def ldv(b, off):
                return pltpu.make_async_copy(
                    vals[v].at[pl.ds(off, K), pl.ds(col, 128)], vb[b], lsem[b])

            def addv(b):
                return pltpu.make_async_copy(vb[b], acc.at[ib[b]], asem[b])

            def body(i, carry):
                cs = []
                for b in range(nbuf):
                    cid = (i * nbuf + b) * NS + s
                    cs.append((active & (cid < n_chunks), cid * K))
                for b in range(nbuf):
                    pred, off = cs[b]

                    @pl.when(pred)
                    def _(b=b, off=off):
                        ldi(b, off).start()
                        ldv(b, off).start()

                for b in range(nbuf):
                    pred, off = cs[b]

                    @pl.when(pred)
                    def _(b=b, off=off):
                        ldi(b, off).wait()
                        ldv(b, off).wait()
                        addv(b).start(add=True)

                for b in range(nbuf):
                    pred, off = cs[b]

                    @pl.when(pred)
                    def _(b=b):
                        addv(b).wait()

                return carry

            lax.fori_loop(0, rounds, body, 0)
            plsc.subcore_barrier()

            @pl.when(active & (s < 15))
            def _():
                pltpu.sync_copy(acc.at[pl.ds(r0, r_lo)],
                                outs[v].at[pl.ds(r0, r_lo), pl.ds(col, 128)])

            @pl.when(active & (s == 15))
            def _():
                pltpu.sync_copy(acc.at[pl.ds(15 * r_lo, r_hi)],
                                outs[v].at[pl.ds(15 * r_lo, r_hi), pl.ds(col, 128)])

            plsc.subcore_barrier()

    res = k(*vals_list, idx, zeros)
    return list(res) if isinstance(res, (list, tuple)) else [res]


def _mlp_refs(x, wrefs):
    h = x * wrefs[0][...]
    for w in wrefs[1:]:
        h = jnp.maximum(h, 0.0)
        h = jnp.dot(h, w[...], preferred_element_type=jnp.float32)
    return h


def _full(shape):
    return pl.BlockSpec(shape, lambda i: tuple(0 for _ in shape))


def _blk(be, d):
    return pl.BlockSpec((be, d), lambda i: (i, 0))


def _tc_edge(emb, sh, g, ws, Bm, be):
    N, D = g.shape
    nw = len(ws)

    def body(*refs):
        emb_ref, sh_ref, g_ref = refs[:3]
        wrefs = refs[3:3 + nw]
        B_ref = refs[3 + nw]
        out_ref = refs[3 + nw + 1]
        w = _mlp_refs(emb_ref[...], wrefs)
        shB = jnp.dot(sh_ref[...], B_ref[...], preferred_element_type=jnp.float32)
        out_ref[...] = w * g_ref[...] * shB

    return pl.pallas_call(
        body,
        grid=(N // be,),
        in_specs=[_blk(be, 1), _blk(be, 9), _blk(be, D)]
        + [_full(w.shape) for w in ws] + [_full(Bm.shape)],
        out_specs=_blk(be, D),
        out_shape=jax.ShapeDtypeStruct((N, D), jnp.float32),
    )(emb, sh, g, *ws, Bm)


def _tc_node_mm(x, mats, br):
    S, Din = x.shape

    def body(*refs):
        x_ref = refs[0]
        m_refs = refs[1:1 + len(mats)]
        out_refs = refs[1 + len(mats):]
        xv = x_ref[...]
        for m_ref, o_ref in zip(m_refs, out_refs):
            o_ref[...] = jnp.dot(xv, m_ref[...], preferred_element_type=jnp.float32)

    return pl.pallas_call(
        body,
        grid=(S // br,),
        in_specs=[_blk(br, Din)] + [_full(m.shape) for m in mats],
        out_specs=[_blk(br, m.shape[1]) for m in mats],
        out_shape=[jax.ShapeDtypeStruct((S, m.shape[1]), jnp.float32) for m in mats],
    )(x, *mats)


def _tc_edge_ch(emb, sh, g, wsC, wsH, BC, BH, be):
    N, D = g.shape

    def body(*refs):
        emb_ref, sh_ref, g_ref = refs[:3]
        i = 3
        wC = refs[i:i + 4]; i += 4
        wH = refs[i:i + 4]; i += 4
        BC_ref, BH_ref, out_ref = refs[i:i + 3]
        shv = sh_ref[...]
        ev = emb_ref[...]
        mC = _mlp_refs(ev, wC)
        mH = _mlp_refs(ev, wH)
        shBC = jnp.dot(shv, BC_ref[...], preferred_element_type=jnp.float32)
        shBH = jnp.dot(shv, BH_ref[...], preferred_element_type=jnp.float32)
        out_ref[...] = g_ref[...] * (mC * shBC + mH * shBH)

    return pl.pallas_call(
        body,
        grid=(N // be,),
        in_specs=[_blk(be, 1), _blk(be, 9), _blk(be, D)]
        + [_full(w.shape) for w in wsC] + [_full(w.shape) for w in wsH]
        + [_full(BC.shape), _full(BH.shape)],
        out_specs=_blk(be, D),
        out_shape=jax.ShapeDtypeStruct((N, D), jnp.float32),
    )(emb, sh, g, *wsC, *wsH, BC, BH)


def _tc_bond(emb_b, sh_b, ga, gb, ws_bond, ws_x, Ax, Bx, blk0, nblk, be):
    Dx = Ax.shape[1]
    nb, nx = len(ws_bond), len(ws_x)

    def off(d):
        return pl.BlockSpec((be, d), lambda i: (blk0 + i, 0))

    def body(*refs):
        emb_ref, sh_ref, ga_ref, gb_ref = refs[:4]
        i = 4
        wb = refs[i:i + nb]; i += nb
        wx = refs[i:i + nx]; i += nx
        A_ref, B_ref, out_ref, out2_ref = refs[i:i + 4]
        ev = emb_ref[...]
        hf = _mlp_refs(ev, wb) * ga_ref[...] * gb_ref[...]
        mx = _mlp_refs(ev, wx)
        hA = jnp.dot(hf, A_ref[...], preferred_element_type=jnp.float32)
        sB = jnp.dot(sh_ref[...], B_ref[...], preferred_element_type=jnp.float32)
        res = mx * hA * sB
        out_ref[...] = res[:, :Dx // 2]
        out2_ref[...] = res[:, Dx // 2:]

    h = Dx // 2
    return pl.pallas_call(
        body,
        grid=(nblk,),
        in_specs=[off(1), off(9), off(ga.shape[1]), off(gb.shape[1])]
        + [_full(w.shape) for w in ws_bond] + [_full(w.shape) for w in ws_x]
        + [_full(Ax.shape), _full(Bx.shape)],
        out_specs=[_blk(be, h), _blk(be, h)],
        out_shape=[jax.ShapeDtypeStruct((nblk * be, h), jnp.float32),
                   jax.ShapeDtypeStruct((nblk * be, h), jnp.float32)],
    )(emb_b, sh_b, ga, gb, *ws_bond, *ws_x, Ax, Bx)


def kernel(sh, emb, f_in, edge_src, edge_dst, num_nodes, num_neighbors,
           HH_ind, CC_ind, CH_ind, fc1, fc2, fc_bond, fcHH, fcCC, fcCH, fcC, fcH,
           A1, B1, A2, B2, Ab, Bb, AHH, BHH, ACC, BCC, ACH, BCH, AC, BC, AH, BH):
    S = f_in.shape[0]
    E = sh.shape[0]
    NB = HH_ind.shape[0]
    NBP = 122880

    inv = 1.0 / jnp.sqrt(jnp.asarray(num_neighbors, jnp.float32))
    w1l = _pad2(fc1[-1] * inv, 16, 128)
    w2l = _pad2(fc2[-1] * inv, 16, 256)
    wCl = _pad2(fcC[-1] * inv, 16, 512)
    wHl = jnp.pad(fcH[-1] * inv, ((0, 0), (392, 70)))
    BHe = jnp.pad(BH, ((0, 0), (392, 70)))

    srcf = edge_src.astype(jnp.float32).reshape(E, 1)
    dstf = edge_dst.astype(jnp.float32).reshape(E, 1)
    meta = jnp.pad(jnp.concatenate([srcf, dstf, emb, sh], axis=1),
                   ((0, 0), (0, 116)))
    inds = jnp.concatenate(
        [HH_ind, CC_ind, CH_ind, jnp.zeros((E - 3 * NB,), jnp.int32)])

    (p1,) = _tc_node_mm(f_in, [_pad2(A1, 2, 128)], 2000)
    (g1,) = _sc_gather_multi([(p1, edge_src)], nbuf=3)
    ef1 = _tc_edge(emb, sh, g1, list(fc1[:-1]) + [w1l], _pad2(B1, 9, 128), 1600)
    (nf1,) = _sc_scatter_multi([ef1], edge_dst, S)

    (p2,) = _tc_node_mm(nf1, [_pad2(A2, 128, 256)], 2000)
    (g2,) = _sc_gather_multi([(p2, edge_src)], nbuf=3)
    ef2 = _tc_edge(emb, sh, g2, list(fc2[:-1]) + [w2l], _pad2(B2, 9, 256), 1600)
    (nf2,) = _sc_scatter_multi([ef2], edge_dst, S)

    ACHw = _pad2(jnp.concatenate([AC, AH], axis=1), 256, 512)
    TA, TB, TCHt = _tc_node_mm(
        nf2, [_pad2(Ab, 256, 256), _pad2(Bb, 256, 256), ACHw], 2000)

    (gCH,) = _sc_gather_multi([(TCHt, edge_src)], K=64, nbuf=3)
    (metag,) = _sc_gather_multi([(meta, inds[:NBP])], nbuf=4)
    edge_CH = _tc_edge_ch(
        emb, sh, gCH,
        list(fcC[:-1]) + [wCl], list(fcH[:-1]) + [wHl],
        _pad2(BC, 9, 512), BHe, 1600)
    (node_CH,) = _sc_scatter_multi([edge_CH], edge_dst, S)
    node_C = node_CH[:, :392]
    node_H = node_CH[:, 392:442]

    b_src = metag[:NBP, 0].astype(jnp.int32)
    b_dst = metag[:NBP, 1].astype(jnp.int32)
    emb_b = metag[:NBP, 2:3]
    sh_b = metag[:NBP, 3:12]
    gA, gB = _sc_gather_multi([(TA, b_src), (TB, b_dst)], K=48, nbuf=4)

    be = 2000
    nblk = NB // be
    wsb = list(fc_bond[:-1]) + [_pad2(fc_bond[-1], 16, 256)]
    eHH0, eHH1 = _tc_bond(emb_b, sh_b, gA, gB, wsb, fcHH,
                          _pad2(AHH, 256, 50), BHH, 0 * nblk, nblk, be)
    eCC0, eCC1 = _tc_bond(emb_b, sh_b, gA, gB, wsb, fcCC,
                          _pad2(ACC, 256, 392), BCC, 1 * nblk, nblk, be)
    eCH0, eCH1 = _tc_bond(emb_b, sh_b, gA, gB, wsb, fcCH,
                          _pad2(ACH, 256, 140), BCH, 2 * nblk, nblk, be)

    hC, hH = 196, 25
    return (node_H[:, :hH], node_C[:, :hC], eHH0, eCH0, eCC0,
            node_H[:, hH:2 * hH], node_C[:, hC:2 * hC], eHH1, eCH1, eCC1)


# speedup vs baseline: 2.2011x; 1.0721x over previous
import functools

import jax
import jax.numpy as jnp
from jax import lax
from jax.experimental import pallas as pl
from jax.experimental.pallas import tpu as pltpu
from jax.experimental.pallas import tpu_sc as plsc

NC = 2
NS = 16
NW = NC * NS
K = 128


def _pad2(m, rows, cols):
    return jnp.pad(m, ((0, rows - m.shape[0]), (0, cols - m.shape[1])))


def _sc_gather_multi(pairs, K=128, nbuf=2):
    N = pairs[0][1].shape[0]
    n_chunks = N // K
    iters = (n_chunks + NW - 1) // NW
    rounds = (iters + nbuf - 1) // nbuf
    n_t = len(pairs)
    uidx, uslot = [], []
    for _, ix in pairs:
        for u, ux in enumerate(uidx):
            if ux is ix:
                uslot.append(u)
                break
        else:
            uslot.append(len(uidx))
            uidx.append(ix)
    n_u = len(uidx)
    mesh = plsc.VectorSubcoreMesh(core_axis_name="c", subcore_axis_name="s")

    scratch = (
        [pltpu.VMEM((K,), jnp.int32) for _ in range(n_u * nbuf)]
        + [pltpu.VMEM((K, t.shape[1]), jnp.float32)
           for t, _ in pairs for _ in range(nbuf)]
        + [pltpu.SemaphoreType.DMA for _ in range(2 * nbuf)]
    )

    @functools.partial(
        pl.kernel,
        mesh=mesh,
        out_type=[jax.ShapeDtypeStruct((N, t.shape[1]), jnp.float32)
                  for t, _ in pairs],
        scratch_types=scratch,
    )
    def k(*refs):
        tabs = refs[0:n_t]
        idxs = refs[n_t:n_t + n_u]
        outs = refs[n_t + n_u:n_t + n_u + n_t]
        sc = refs[n_t + n_u + n_t:]
        ib = [sc[u * nbuf:(u + 1) * nbuf] for u in range(n_u)]
        sc = sc[n_u * nbuf:]
        rb = [sc[j * nbuf:(j + 1) * nbuf] for j in range(n_t)]
        sc = sc[n_t * nbuf:]
        gsem, ssem = sc[:nbuf], sc[nbuf:2 * nbuf]
        wid = lax.axis_index("s") * NC + lax.axis_index("c")

        def gath(j, b):
            return pltpu.make_async_copy(
                tabs[j].at[ib[uslot[j]][b]], rb[j][b], gsem[b])

        def stor(j, b, off):
            return pltpu.make_async_copy(
                rb[j][b], outs[j].at[pl.ds(off, K)], ssem[b])

        def body(r, carry):
            cs = []
            for b in range(nbuf):
                cid = (r * nbuf + b) * NW + wid
                cs.append((cid < n_chunks, cid * K))
            for b in range(nbuf):
                pred, off = cs[b]

                @pl.when(pred)
                def _(b=b, off=off):
                    for u in range(n_u):
                        pltpu.sync_copy(idxs[u].at[pl.ds(off, K)], ib[u][b])
                    for j in range(n_t):
                        gath(j, b).start()

            for b in range(nbuf):
                pred, off = cs[b]

                @pl.when(pred)
                def _(b=b, off=off):
                    for j in range(n_t):
                        gath(j, b).wait()
                    for j in range(n_t):
                        stor(j, b, off).start()

            for b in range(nbuf):
                pred, off = cs[b]

                @pl.when(pred)
                def _(b=b, off=off):
                    for j in range(n_t):
                        stor(j, b, off).wait()

            return carry

        lax.fori_loop(0, rounds, body, 0)

    res = k(*[t for t, _ in pairs], *uidx)
    return list(res) if isinstance(res, (list, tuple)) else [res]


def _sc_scatter_multi(vals_list, idx, S):
    N = idx.shape[0]
    n_chunks = N // K
    iters = (n_chunks + NS - 1) // NS
    n_v = len(vals_list)
    zeros = jnp.zeros((S, 128), jnp.float32)
    mesh = plsc.VectorSubcoreMesh(core_axis_name="c", subcore_axis_name="s")
    phases = []
    for v, a in enumerate(vals_list):
        for cb in range(0, a.shape[1], 256):
            phases.append((v, cb))
    r_lo, r_hi = (S // 16) // 8 * 8, S - 15 * ((S // 16) // 8 * 8)

    nbuf = 3
    rounds = (iters + nbuf - 1) // nbuf

    @functools.partial(
        pl.kernel,
        mesh=mesh,
        out_type=[jax.ShapeDtypeStruct((S, a.shape[1]), jnp.float32)
                  for a in vals_list],
        scratch_types=(
            [pltpu.VMEM((K,), jnp.int32) for _ in range(nbuf)]
            + [pltpu.VMEM((K, 128), jnp.float32) for _ in range(nbuf)]
            + [pltpu.VMEM_SHARED((S, 128), jnp.float32)]
            + [pltpu.SemaphoreType.DMA for _ in range(2 * nbuf)]
        ),
    )
    def k(*refs):
        vals = refs[0:n_v]
        idx_hbm = refs[n_v]
        zeros_hbm = refs[n_v + 1]
        outs = refs[n_v + 2:n_v + 2 + n_v]
        sc = refs[n_v + 2 + n_v:]
        ib, vb = sc[:nbuf], sc[nbuf:2 * nbuf]
        acc = sc[2 * nbuf]
        lsem = sc[2 * nbuf + 1:2 * nbuf + 1 + nbuf]
        asem = sc[2 * nbuf + 1 + nbuf:2 * nbuf + 1 + 2 * nbuf]
        c = lax.axis_index("c")
        s = lax.axis_index("s")
        r0 = s * r_lo

        for v, cb in phases:
            D = vals_list[v].shape[1]
            col = cb + c * 128
            active = col < D

            @pl.when(active & (s < 15))
            def _():
                pltpu.sync_copy(zeros_hbm.at[pl.ds(r0, r_lo)],
                                acc.at[pl.ds(r0, r_lo)])

            @pl.when(active & (s == 15))
            def _():
                pltpu.sync_copy(zeros_hbm.at[pl.ds(15 * r_lo, r_hi)],
                                acc.at[pl.ds(15 * r_lo, r_hi)])

            plsc.subcore_barrier()

            def ldi(b, off):
                return pltpu.make_async_copy(
                    idx_hbm.at[pl.ds(off, K)], ib[b], lsem[b])

            def ldv(b, off):
                return pltpu.make_async_copy(
                    vals[v].at[pl.ds(off, K), pl.ds(col, 128)], vb[b], lsem[b])

            def addv(b):
                return pltpu.make_async_copy(vb[b], acc.at[ib[b]], asem[b])

            def body(i, carry):
                cs = []
                for b in range(nbuf):
                    cid = (i * nbuf + b) * NS + s
                    cs.append((active & (cid < n_chunks), cid * K))
                for b in range(nbuf):
                    pred, off = cs[b]

                    @pl.when(pred)
                    def _(b=b, off=off):
                        ldi(b, off).start()
                        ldv(b, off).start()

                for b in range(nbuf):
                    pred, off = cs[b]

                    @pl.when(pred)
                    def _(b=b, off=off):
                        ldi(b, off).wait()
                        ldv(b, off).wait()
                        addv(b).start(add=True)

                for b in range(nbuf):
                    pred, off = cs[b]

                    @pl.when(pred)
                    def _(b=b):
                        addv(b).wait()

                return carry

            lax.fori_loop(0, rounds, body, 0)
            plsc.subcore_barrier()

            @pl.when(active & (s < 15))
            def _():
                pltpu.sync_copy(acc.at[pl.ds(r0, r_lo)],
                                outs[v].at[pl.ds(r0, r_lo), pl.ds(col, 128)])

            @pl.when(active & (s == 15))
            def _():
                pltpu.sync_copy(acc.at[pl.ds(15 * r_lo, r_hi)],
                                outs[v].at[pl.ds(15 * r_lo, r_hi), pl.ds(col, 128)])

            plsc.subcore_barrier()

    res = k(*vals_list, idx, zeros)
    return list(res) if isinstance(res, (list, tuple)) else [res]


def _mlp_refs(x, wrefs):
    h = x * wrefs[0][...]
    for w in wrefs[1:]:
        h = jnp.maximum(h, 0.0)
        h = jnp.dot(h, w[...], preferred_element_type=jnp.float32)
    return h


def _full(shape):
    return pl.BlockSpec(shape, lambda i: tuple(0 for _ in shape))


def _blk(be, d):
    return pl.BlockSpec((be, d), lambda i: (i, 0))


def _tc_edge(emb, sh, g, ws, Bm, be):
    N, D = g.shape
    nw = len(ws)

    def body(*refs):
        emb_ref, sh_ref, g_ref = refs[:3]
        wrefs = refs[3:3 + nw]
        B_ref = refs[3 + nw]
        out_ref = refs[3 + nw + 1]
        w = _mlp_refs(emb_ref[...], wrefs)
        shB = jnp.dot(sh_ref[...], B_ref[...], preferred_element_type=jnp.float32)
        out_ref[...] = w * g_ref[...] * shB

    return pl.pallas_call(
        body,
        grid=(N // be,),
        in_specs=[_blk(be, 1), _blk(be, 9), _blk(be, D)]
        + [_full(w.shape) for w in ws] + [_full(Bm.shape)],
        out_specs=_blk(be, D),
        out_shape=jax.ShapeDtypeStruct((N, D), jnp.float32),
    )(emb, sh, g, *ws, Bm)


def _tc_node_mm(x, mats, br):
    S, Din = x.shape

    def body(*refs):
        x_ref = refs[0]
        m_refs = refs[1:1 + len(mats)]
        out_refs = refs[1 + len(mats):]
        xv = x_ref[...]
        for m_ref, o_ref in zip(m_refs, out_refs):
            o_ref[...] = jnp.dot(xv, m_ref[...], preferred_element_type=jnp.float32)

    return pl.pallas_call(
        body,
        grid=(S // br,),
        in_specs=[_blk(br, Din)] + [_full(m.shape) for m in mats],
        out_specs=[_blk(br, m.shape[1]) for m in mats],
        out_shape=[jax.ShapeDtypeStruct((S, m.shape[1]), jnp.float32) for m in mats],
    )(x, *mats)


def _bf16_round(u):
    return (u + 0x7FFF + ((u >> 16) & 1)) >> 16


def _tc_node_mm_pack(x, M, br):
    S, Din = x.shape
    H = M.shape[1] // 2

    def body(x_ref, m_ref, o_ref):
        t = jnp.dot(x_ref[...], m_ref[...], preferred_element_type=jnp.float32)
        ul = _bf16_round(lax.bitcast_convert_type(t[:, :H], jnp.uint32))
        uh = _bf16_round(lax.bitcast_convert_type(t[:, H:], jnp.uint32))
        o_ref[...] = lax.bitcast_convert_type(ul | (uh << 16), jnp.float32)

    return pl.pallas_call(
        body,
        grid=(S // br,),
        in_specs=[_blk(br, Din), _full(M.shape)],
        out_specs=_blk(br, H),
        out_shape=jax.ShapeDtypeStruct((S, H), jnp.float32),
    )(x, M)


def _unpack_lo_hi(g):
    u = lax.bitcast_convert_type(g, jnp.uint32)
    lo = lax.bitcast_convert_type(u << 16, jnp.float32)
    hi = lax.bitcast_convert_type(u & jnp.uint32(0xFFFF0000), jnp.float32)
    return lo, hi


def _tc_edge_ch(emb, sh, g, wsC, wsH, BC, BH, be):
    N, H = g.shape

    def body(*refs):
        emb_ref, sh_ref, g_ref = refs[:3]
        i = 3
        wC = refs[i:i + 4]; i += 4
        wH = refs[i:i + 4]; i += 4
        BC_ref, BH_ref, lo_ref, hi_ref = refs[i:i + 4]
        shv = sh_ref[...]
        ev = emb_ref[...]
        mC = _mlp_refs(ev, wC)
        mH = _mlp_refs(ev, wH)
        shBC = jnp.dot(shv, BC_ref[...], preferred_element_type=jnp.float32)
        shBH = jnp.dot(shv, BH_ref[...], preferred_element_type=jnp.float32)
        m = mC * shBC + mH * shBH
        glo, ghi = _unpack_lo_hi(g_ref[...])
        lo_ref[...] = glo * m[:, :H]
        hi_ref[...] = ghi * m[:, H:]

    return pl.pallas_call(
        body,
        grid=(N // be,),
        in_specs=[_blk(be, 1), _blk(be, 9), _blk(be, H)]
        + [_full(w.shape) for w in wsC] + [_full(w.shape) for w in wsH]
        + [_full(BC.shape), _full(BH.shape)],
        out_specs=[_blk(be, H), _blk(be, H)],
        out_shape=[jax.ShapeDtypeStruct((N, H), jnp.float32),
                   jax.ShapeDtypeStruct((N, H), jnp.float32)],
    )(emb, sh, g, *wsC, *wsH, BC, BH)


def _tc_bond(emb_b, sh_b, ga, gb, ws_bond, ws_x, Ax, Bx, blk0, nblk, be):
    Dx = Ax.shape[1]
    nb, nx = len(ws_bond), len(ws_x)

    def off(d):
        return pl.BlockSpec((be, d), lambda i: (blk0 + i, 0))

    def body(*refs):
        emb_ref, sh_ref, ga_ref, gb_ref = refs[:4]
        i = 4
        wb = refs[i:i + nb]; i += nb
        wx = refs[i:i + nx]; i += nx
        A_ref, B_ref, out_ref, out2_ref = refs[i:i + 4]
        ev = emb_ref[...]
        hf = _mlp_refs(ev, wb) * ga_ref[...] * gb_ref[...]
        mx = _mlp_refs(ev, wx)
        hA = jnp.dot(hf, A_ref[...], preferred_element_type=jnp.float32)
        sB = jnp.dot(sh_ref[...], B_ref[...], preferred_element_type=jnp.float32)
        res = mx * hA * sB
        out_ref[...] = res[:, :Dx // 2]
        out2_ref[...] = res[:, Dx // 2:]

    h = Dx // 2
    return pl.pallas_call(
        body,
        grid=(nblk,),
        in_specs=[off(1), off(9), off(ga.shape[1]), off(gb.shape[1])]
        + [_full(w.shape) for w in ws_bond] + [_full(w.shape) for w in ws_x]
        + [_full(Ax.shape), _full(Bx.shape)],
        out_specs=[_blk(be, h), _blk(be, h)],
        out_shape=[jax.ShapeDtypeStruct((nblk * be, h), jnp.float32),
                   jax.ShapeDtypeStruct((nblk * be, h), jnp.float32)],
    )(emb_b, sh_b, ga, gb, *ws_bond, *ws_x, Ax, Bx)


def kernel(sh, emb, f_in, edge_src, edge_dst, num_nodes, num_neighbors,
           HH_ind, CC_ind, CH_ind, fc1, fc2, fc_bond, fcHH, fcCC, fcCH, fcC, fcH,
           A1, B1, A2, B2, Ab, Bb, AHH, BHH, ACC, BCC, ACH, BCH, AC, BC, AH, BH):
    S = f_in.shape[0]
    E = sh.shape[0]
    NB = HH_ind.shape[0]
    NBP = 122880

    inv = 1.0 / jnp.sqrt(jnp.asarray(num_neighbors, jnp.float32))
    w1l = _pad2(fc1[-1] * inv, 16, 128)
    w2l = _pad2(fc2[-1] * inv, 16, 256)
    wCl = _pad2(fcC[-1] * inv, 16, 512)
    wHl = jnp.pad(fcH[-1] * inv, ((0, 0), (392, 70)))
    BHe = jnp.pad(BH, ((0, 0), (392, 70)))

    srcf = edge_src.astype(jnp.float32).reshape(E, 1)
    dstf = edge_dst.astype(jnp.float32).reshape(E, 1)
    meta = jnp.pad(jnp.concatenate([srcf, dstf, emb, sh], axis=1),
                   ((0, 0), (0, 116)))
    inds = jnp.concatenate(
        [HH_ind, CC_ind, CH_ind, jnp.zeros((E - 3 * NB,), jnp.int32)])

    (p1,) = _tc_node_mm(f_in, [_pad2(A1, 2, 128)], 2000)
    (g1,) = _sc_gather_multi([(p1, edge_src)], nbuf=3)
    ef1 = _tc_edge(emb, sh, g1, list(fc1[:-1]) + [w1l], _pad2(B1, 9, 128), 1600)
    (nf1,) = _sc_scatter_multi([ef1], edge_dst, S)

    (p2,) = _tc_node_mm(nf1, [_pad2(A2, 128, 256)], 2000)
    (g2,) = _sc_gather_multi([(p2, edge_src)], nbuf=3)
    ef2 = _tc_edge(emb, sh, g2, list(fc2[:-1]) + [w2l], _pad2(B2, 9, 256), 1600)
    (nf2,) = _sc_scatter_multi([ef2], edge_dst, S)

    ACHw = _pad2(jnp.concatenate([AC, AH], axis=1), 256, 512)
    TA, TB = _tc_node_mm(
        nf2, [_pad2(Ab, 256, 256), _pad2(Bb, 256, 256)], 2000)
    TCHt = _tc_node_mm_pack(nf2, ACHw, 2000)

    (gCH,) = _sc_gather_multi([(TCHt, edge_src)], nbuf=3)
    (metag,) = _sc_gather_multi([(meta, inds[:NBP])], nbuf=4)
    eCH_lo, eCH_hi = _tc_edge_ch(
        emb, sh, gCH,
        list(fcC[:-1]) + [wCl], list(fcH[:-1]) + [wHl],
        _pad2(BC, 9, 512), BHe, 1600)
    n_lo, n_hi = _sc_scatter_multi([eCH_lo, eCH_hi], edge_dst, S)
    node_C = jnp.concatenate([n_lo, n_hi[:, :136]], axis=1)
    node_H = n_hi[:, 136:186]

    b_src = metag[:NBP, 0].astype(jnp.int32)
    b_dst = metag[:NBP, 1].astype(jnp.int32)
    emb_b = metag[:NBP, 2:3]
    sh_b = metag[:NBP, 3:12]
    gA, gB = _sc_gather_multi([(TA, b_src), (TB, b_dst)], K=48, nbuf=4)

    be = 2000
    nblk = NB // be
    wsb = list(fc_bond[:-1]) + [_pad2(fc_bond[-1], 16, 256)]
    eHH0, eHH1 = _tc_bond(emb_b, sh_b, gA, gB, wsb, fcHH,
                          _pad2(AHH, 256, 50), BHH, 0 * nblk, nblk, be)
    eCC0, eCC1 = _tc_bond(emb_b, sh_b, gA, gB, wsb, fcCC,
                          _pad2(ACC, 256, 392), BCC, 1 * nblk, nblk, be)
    eCH0, eCH1 = _tc_bond(emb_b, sh_b, gA, gB, wsb, fcCH,
                          _pad2(ACH, 256, 140), BCH, 2 * nblk, nblk, be)

    hC, hH = 196, 25
    return (node_H[:, :hH], node_C[:, :hC], eHH0, eCH0, eCC0,
            node_H[:, hH:2 * hH], node_C[:, hC:2 * hC], eHH1, eCH1, eCC1)


# speedup vs baseline: 2.2821x; 1.0368x over previous
import functools

import jax
import jax.numpy as jnp
from jax import lax
from jax.experimental import pallas as pl
from jax.experimental.pallas import tpu as pltpu
from jax.experimental.pallas import tpu_sc as plsc

NC = 2
NS = 16
NW = NC * NS
K = 128


def _pad2(m, rows, cols):
    return jnp.pad(m, ((0, rows - m.shape[0]), (0, cols - m.shape[1])))


def _sc_gather_multi(pairs, K=128, nbuf=2):
    N = pairs[0][1].shape[0]
    n_chunks = N // K
    iters = (n_chunks + NW - 1) // NW
    rounds = (iters + nbuf - 1) // nbuf
    n_t = len(pairs)
    uidx, uslot = [], []
    for _, ix in pairs:
        for u, ux in enumerate(uidx):
            if ux is ix:
                uslot.append(u)
                break
        else:
            uslot.append(len(uidx))
            uidx.append(ix)
    n_u = len(uidx)
    mesh = plsc.VectorSubcoreMesh(core_axis_name="c", subcore_axis_name="s")

    scratch = (
        [pltpu.VMEM((K,), jnp.int32) for _ in range(n_u * nbuf)]
        + [pltpu.VMEM((K, t.shape[1]), jnp.float32)
           for t, _ in pairs for _ in range(nbuf)]
        + [pltpu.SemaphoreType.DMA for _ in range(2 * nbuf)]
    )

    @functools.partial(
        pl.kernel,
        mesh=mesh,
        out_type=[jax.ShapeDtypeStruct((N, t.shape[1]), jnp.float32)
                  for t, _ in pairs],
        scratch_types=scratch,
    )
    def k(*refs):
        tabs = refs[0:n_t]
        idxs = refs[n_t:n_t + n_u]
        outs = refs[n_t + n_u:n_t + n_u + n_t]
        sc = refs[n_t + n_u + n_t:]
        ib = [sc[u * nbuf:(u + 1) * nbuf] for u in range(n_u)]
        sc = sc[n_u * nbuf:]
        rb = [sc[j * nbuf:(j + 1) * nbuf] for j in range(n_t)]
        sc = sc[n_t * nbuf:]
        gsem, ssem = sc[:nbuf], sc[nbuf:2 * nbuf]
        wid = lax.axis_index("s") * NC + lax.axis_index("c")

        def gath(j, b):
            return pltpu.make_async_copy(
                tabs[j].at[ib[uslot[j]][b]], rb[j][b], gsem[b])

        def stor(j, b, off):
            return pltpu.make_async_copy(
                rb[j][b], outs[j].at[pl.ds(off, K)], ssem[b])

        def body(r, carry):
            cs = []
            for b in range(nbuf):
                cid = (r * nbuf + b) * NW + wid
                cs.append((cid < n_chunks, cid * K))
            for b in range(nbuf):
                pred, off = cs[b]

                @pl.when(pred)
                def _(b=b, off=off):
                    for u in range(n_u):
                        pltpu.sync_copy(idxs[u].at[pl.ds(off, K)], ib[u][b])
                    for j in range(n_t):
                        gath(j, b).start()

            for b in range(nbuf):
                pred, off = cs[b]

                @pl.when(pred)
                def _(b=b, off=off):
                    for j in range(n_t):
                        gath(j, b).wait()
                    for j in range(n_t):
                        stor(j, b, off).start()

            for b in range(nbuf):
                pred, off = cs[b]

                @pl.when(pred)
                def _(b=b, off=off):
                    for j in range(n_t):
                        stor(j, b, off).wait()

            return carry

        lax.fori_loop(0, rounds, body, 0)

    res = k(*[t for t, _ in pairs], *uidx)
    return list(res) if isinstance(res, (list, tuple)) else [res]


def _sc_scatter_multi(vals_list, idx, S):
    N = idx.shape[0]
    n_chunks = N // K
    iters = (n_chunks + NS - 1) // NS
    n_v = len(vals_list)
    zeros = jnp.zeros((S, 128), jnp.float32)
    mesh = plsc.VectorSubcoreMesh(core_axis_name="c", subcore_axis_name="s")
    phases = []
    for v, a in enumerate(vals_list):
        for cb in range(0, a.shape[1], 256):
            phases.append((v, cb))
    r_lo, r_hi = (S // 16) // 8 * 8, S - 15 * ((S // 16) // 8 * 8)

    nbuf = 3
    rounds = (iters + nbuf - 1) // nbuf

    @functools.partial(
        pl.kernel,
        mesh=mesh,
        out_type=[jax.ShapeDtypeStruct((S, a.shape[1]), jnp.float32)
                  for a in vals_list],
        scratch_types=(
            [pltpu.VMEM((K,), jnp.int32) for _ in range(nbuf)]
            + [pltpu.VMEM((K, 128), jnp.float32) for _ in range(nbuf)]
            + [pltpu.VMEM_SHARED((S, 128), jnp.float32)]
            + [pltpu.SemaphoreType.DMA for _ in range(2 * nbuf)]
        ),
    )
    def k(*refs):
        vals = refs[0:n_v]
        idx_hbm = refs[n_v]
        zeros_hbm = refs[n_v + 1]
        outs = refs[n_v + 2:n_v + 2 + n_v]
        sc = refs[n_v + 2 + n_v:]
        ib, vb = sc[:nbuf], sc[nbuf:2 * nbuf]
        acc = sc[2 * nbuf]
        lsem = sc[2 * nbuf + 1:2 * nbuf + 1 + nbuf]
        asem = sc[2 * nbuf + 1 + nbuf:2 * nbuf + 1 + 2 * nbuf]
        c = lax.axis_index("c")
        s = lax.axis_index("s")
        r0 = s * r_lo

        for v, cb in phases:
            D = vals_list[v].shape[1]
            col = cb + c * 128
            active = col < D

            @pl.when(active & (s < 15))
            def _():
                pltpu.sync_copy(zeros_hbm.at[pl.ds(r0, r_lo)],
                                acc.at[pl.ds(r0, r_lo)])

            @pl.when(active & (s == 15))
            def _():
                pltpu.sync_copy(zeros_hbm.at[pl.ds(15 * r_lo, r_hi)],
                                acc.at[pl.ds(15 * r_lo, r_hi)])

            plsc.subcore_barrier()

            def ldi(b, off):
                return pltpu.make_async_copy(
                    idx_hbm.at[pl.ds(off, K)], ib[b], lsem[b])

            def ldv(b, off):
                return pltpu.make_async_copy(
                    vals[v].at[pl.ds(off, K), pl.ds(col, 128)], vb[b], lsem[b])

            def addv(b):
                return pltpu.make_async_copy(vb[b], acc.at[ib[b]], asem[b])

            def body(i, carry):
                cs = []
                for b in range(nbuf):
                    cid = (i * nbuf + b) * NS + s
                    cs.append((active & (cid < n_chunks), cid * K))
                for b in range(nbuf):
                    pred, off = cs[b]

                    @pl.when(pred)
                    def _(b=b, off=off):
                        ldi(b, off).start()
                        ldv(b, off).start()

                for b in range(nbuf):
                    pred, off = cs[b]

                    @pl.when(pred)
                    def _(b=b, off=off):
                        ldi(b, off).wait()
                        ldv(b, off).wait()
                        addv(b).start(add=True)

                for b in range(nbuf):
                    pred, off = cs[b]

                    @pl.when(pred)
                    def _(b=b):
                        addv(b).wait()

                return carry

            lax.fori_loop(0, rounds, body, 0)
            plsc.subcore_barrier()

            @pl.when(active & (s < 15))
            def _():
                pltpu.sync_copy(acc.at[pl.ds(r0, r_lo)],
                                outs[v].at[pl.ds(r0, r_lo), pl.ds(col, 128)])

            @pl.when(active & (s == 15))
            def _():
                pltpu.sync_copy(acc.at[pl.ds(15 * r_lo, r_hi)],
                                outs[v].at[pl.ds(15 * r_lo, r_hi), pl.ds(col, 128)])

            plsc.subcore_barrier()

    res = k(*vals_list, idx, zeros)
    return list(res) if isinstance(res, (list, tuple)) else [res]


def _mlp_refs(x, wrefs):
    h = x * wrefs[0][...]
    for w in wrefs[1:]:
        h = jnp.maximum(h, 0.0)
        h = jnp.dot(h, w[...], preferred_element_type=jnp.float32)
    return h


def _full(shape):
    return pl.BlockSpec(shape, lambda i: tuple(0 for _ in shape))


def _blk(be, d):
    return pl.BlockSpec((be, d), lambda i: (i, 0))


def _tc_edge(emb, sh, g, ws, Bm, be):
    N, D = g.shape
    nw = len(ws)

    def body(*refs):
        emb_ref, sh_ref, g_ref = refs[:3]
        wrefs = refs[3:3 + nw]
        B_ref = refs[3 + nw]
        out_ref = refs[3 + nw + 1]
        w = _mlp_refs(emb_ref[...], wrefs)
        shB = jnp.dot(sh_ref[...], B_ref[...], preferred_element_type=jnp.float32)
        out_ref[...] = w * g_ref[...] * shB

    return pl.pallas_call(
        body,
        grid=(N // be,),
        in_specs=[_blk(be, 1), _blk(be, 9), _blk(be, D)]
        + [_full(w.shape) for w in ws] + [_full(Bm.shape)],
        out_specs=_blk(be, D),
        out_shape=jax.ShapeDtypeStruct((N, D), jnp.float32),
    )(emb, sh, g, *ws, Bm)


def _tc_node_mm(x, mats, br):
    S, Din = x.shape

    def body(*refs):
        x_ref = refs[0]
        m_refs = refs[1:1 + len(mats)]
        out_refs = refs[1 + len(mats):]
        xv = x_ref[...]
        for m_ref, o_ref in zip(m_refs, out_refs):
            o_ref[...] = jnp.dot(xv, m_ref[...], preferred_element_type=jnp.float32)

    return pl.pallas_call(
        body,
        grid=(S // br,),
        in_specs=[_blk(br, Din)] + [_full(m.shape) for m in mats],
        out_specs=[_blk(br, m.shape[1]) for m in mats],
        out_shape=[jax.ShapeDtypeStruct((S, m.shape[1]), jnp.float32) for m in mats],
    )(x, *mats)


def _bf16_round(u):
    return (u + 0x7FFF + ((u >> 16) & 1)) >> 16


def _tc_node_mm_pack(x, M, br):
    S, Din = x.shape
    H = M.shape[1] // 2

    def body(x_ref, m_ref, o_ref):
        t = jnp.dot(x_ref[...], m_ref[...], preferred_element_type=jnp.float32)
        ul = _bf16_round(lax.bitcast_convert_type(t[:, :H], jnp.uint32))
        uh = _bf16_round(lax.bitcast_convert_type(t[:, H:], jnp.uint32))
        o_ref[...] = lax.bitcast_convert_type(ul | (uh << 16), jnp.float32)

    return pl.pallas_call(
        body,
        grid=(S // br,),
        in_specs=[_blk(br, Din), _full(M.shape)],
        out_specs=_blk(br, H),
        out_shape=jax.ShapeDtypeStruct((S, H), jnp.float32),
    )(x, M)


def _unpack_lo_hi(g):
    u = lax.bitcast_convert_type(g, jnp.uint32)
    lo = lax.bitcast_convert_type(u << 16, jnp.float32)
    hi = lax.bitcast_convert_type(u & jnp.uint32(0xFFFF0000), jnp.float32)
    return lo, hi


def _tc_edge_ch(emb, sh, g, wsC, wsH, BC, BH, be):
    N, H = g.shape

    def body(*refs):
        emb_ref, sh_ref, g_ref = refs[:3]
        i = 3
        wC = refs[i:i + 4]; i += 4
        wH = refs[i:i + 4]; i += 4
        BC_ref, BH_ref, lo_ref, hi_ref = refs[i:i + 4]
        shv = sh_ref[...]
        ev = emb_ref[...]
        mC = _mlp_refs(ev, wC)
        mH = _mlp_refs(ev, wH)
        shBC = jnp.dot(shv, BC_ref[...], preferred_element_type=jnp.float32)
        shBH = jnp.dot(shv, BH_ref[...], preferred_element_type=jnp.float32)
        m = mC * shBC + mH * shBH
        glo, ghi = _unpack_lo_hi(g_ref[...])
        lo_ref[...] = glo * m[:, :H]
        hi_ref[...] = ghi * m[:, H:]

    return pl.pallas_call(
        body,
        grid=(N // be,),
        in_specs=[_blk(be, 1), _blk(be, 9), _blk(be, H)]
        + [_full(w.shape) for w in wsC] + [_full(w.shape) for w in wsH]
        + [_full(BC.shape), _full(BH.shape)],
        out_specs=[_blk(be, H), _blk(be, H)],
        out_shape=[jax.ShapeDtypeStruct((N, H), jnp.float32),
                   jax.ShapeDtypeStruct((N, H), jnp.float32)],
    )(emb, sh, g, *wsC, *wsH, BC, BH)


def _tc_bond(emb_b, sh_b, ga, gb, ws_bond, ws_x, Ax, Bx, blk0, nblk, be):
    Dx = Ax.shape[1]
    nb, nx = len(ws_bond), len(ws_x)

    def off(d):
        return pl.BlockSpec((be, d), lambda i: (blk0 + i, 0))

    def body(*refs):
        emb_ref, sh_ref, ga_ref, gb_ref = refs[:4]
        i = 4
        wb = refs[i:i + nb]; i += nb
        wx = refs[i:i + nx]; i += nx
        A_ref, B_ref, out_ref, out2_ref = refs[i:i + 4]
        ev = emb_ref[...]
        ga_lo, ga_hi = _unpack_lo_hi(ga_ref[...])
        gb_lo, gb_hi = _unpack_lo_hi(gb_ref[...])
        wbv = _mlp_refs(ev, wb)
        hf_lo = wbv[:, :128] * ga_lo * gb_lo
        hf_hi = wbv[:, 128:] * ga_hi * gb_hi
        mx = _mlp_refs(ev, wx)
        Av = A_ref[...]
        hA = (jnp.dot(hf_lo, Av[:128], preferred_element_type=jnp.float32)
              + jnp.dot(hf_hi, Av[128:], preferred_element_type=jnp.float32))
        sB = jnp.dot(sh_ref[...], B_ref[...], preferred_element_type=jnp.float32)
        res = mx * hA * sB
        out_ref[...] = res[:, :Dx // 2]
        out2_ref[...] = res[:, Dx // 2:]

    h = Dx // 2
    return pl.pallas_call(
        body,
        grid=(nblk,),
        in_specs=[off(1), off(9), off(ga.shape[1]), off(gb.shape[1])]
        + [_full(w.shape) for w in ws_bond] + [_full(w.shape) for w in ws_x]
        + [_full(Ax.shape), _full(Bx.shape)],
        out_specs=[_blk(be, h), _blk(be, h)],
        out_shape=[jax.ShapeDtypeStruct((nblk * be, h), jnp.float32),
                   jax.ShapeDtypeStruct((nblk * be, h), jnp.float32)],
    )(emb_b, sh_b, ga, gb, *ws_bond, *ws_x, Ax, Bx)


def kernel(sh, emb, f_in, edge_src, edge_dst, num_nodes, num_neighbors,
           HH_ind, CC_ind, CH_ind, fc1, fc2, fc_bond, fcHH, fcCC, fcCH, fcC, fcH,
           A1, B1, A2, B2, Ab, Bb, AHH, BHH, ACC, BCC, ACH, BCH, AC, BC, AH, BH):
    S = f_in.shape[0]
    E = sh.shape[0]
    NB = HH_ind.shape[0]
    NBP = 122880

    inv = 1.0 / jnp.sqrt(jnp.asarray(num_neighbors, jnp.float32))
    w1l = _pad2(fc1[-1] * inv, 16, 128)
    w2l = _pad2(fc2[-1] * inv, 16, 256)
    wCl = _pad2(fcC[-1] * inv, 16, 512)
    wHl = jnp.pad(fcH[-1] * inv, ((0, 0), (392, 70)))
    BHe = jnp.pad(BH, ((0, 0), (392, 70)))

    srcf = edge_src.astype(jnp.float32).reshape(E, 1)
    dstf = edge_dst.astype(jnp.float32).reshape(E, 1)
    meta = jnp.pad(jnp.concatenate([srcf, dstf, emb, sh], axis=1),
                   ((0, 0), (0, 116)))
    inds = jnp.concatenate(
        [HH_ind, CC_ind, CH_ind, jnp.zeros((E - 3 * NB,), jnp.int32)])

    (p1,) = _tc_node_mm(f_in, [_pad2(A1, 2, 128)], 2000)
    (g1,) = _sc_gather_multi([(p1, edge_src)], nbuf=3)
    ef1 = _tc_edge(emb, sh, g1, list(fc1[:-1]) + [w1l], _pad2(B1, 9, 128), 1600)
    (nf1,) = _sc_scatter_multi([ef1], edge_dst, S)

    (p2,) = _tc_node_mm(nf1, [_pad2(A2, 128, 256)], 2000)
    (g2,) = _sc_gather_multi([(p2, edge_src)], nbuf=3)
    ef2 = _tc_edge(emb, sh, g2, list(fc2[:-1]) + [w2l], _pad2(B2, 9, 256), 1600)
    (nf2,) = _sc_scatter_multi([ef2], edge_dst, S)

    ACHw = _pad2(jnp.concatenate([AC, AH], axis=1), 256, 512)
    TA = _tc_node_mm_pack(nf2, _pad2(Ab, 256, 256), 2000)
    TB = _tc_node_mm_pack(nf2, _pad2(Bb, 256, 256), 2000)
    TCHt = _tc_node_mm_pack(nf2, ACHw, 2000)

    (gCH,) = _sc_gather_multi([(TCHt, edge_src)], nbuf=3)
    (metag,) = _sc_gather_multi([(meta, inds[:NBP])], nbuf=4)
    eCH_lo, eCH_hi = _tc_edge_ch(
        emb, sh, gCH,
        list(fcC[:-1]) + [wCl], list(fcH[:-1]) + [wHl],
        _pad2(BC, 9, 512), BHe, 1600)
    n_lo, n_hi = _sc_scatter_multi([eCH_lo, eCH_hi], edge_dst, S)
    node_C = jnp.concatenate([n_lo, n_hi[:, :136]], axis=1)
    node_H = n_hi[:, 136:186]

    b_src = metag[:NBP, 0].astype(jnp.int32)
    b_dst = metag[:NBP, 1].astype(jnp.int32)
    emb_b = metag[:NBP, 2:3]
    sh_b = metag[:NBP, 3:12]
    gA, gB = _sc_gather_multi([(TA, b_src), (TB, b_dst)], nbuf=3)

    be = 2000
    nblk = NB // be
    wsb = list(fc_bond[:-1]) + [_pad2(fc_bond[-1], 16, 256)]
    eHH0, eHH1 = _tc_bond(emb_b, sh_b, gA, gB, wsb, fcHH,
                          _pad2(AHH, 256, 50), BHH, 0 * nblk, nblk, be)
    eCC0, eCC1 = _tc_bond(emb_b, sh_b, gA, gB, wsb, fcCC,
                          _pad2(ACC, 256, 392), BCC, 1 * nblk, nblk, be)
    eCH0, eCH1 = _tc_bond(emb_b, sh_b, gA, gB, wsb, fcCH,
                          _pad2(ACH, 256, 140), BCH, 2 * nblk, nblk, be)

    hC, hH = 196, 25
    return (node_H[:, :hH], node_C[:, :hC], eHH0, eCH0, eCC0,
            node_H[:, hH:2 * hH], node_C[:, hC:2 * hC], eHH1, eCH1, eCC1)


# speedup vs baseline: 2.3558x; 1.0323x over previous
import functools

import jax
import jax.numpy as jnp
from jax import lax
from jax.experimental import pallas as pl
from jax.experimental.pallas import tpu as pltpu
from jax.experimental.pallas import tpu_sc as plsc

NC = 2
NS = 16
NW = NC * NS
K = 128


def _pad2(m, rows, cols):
    return jnp.pad(m, ((0, rows - m.shape[0]), (0, cols - m.shape[1])))


def _sc_gather_multi(pairs, K=128, nbuf=2):
    N = pairs[0][1].shape[0]
    n_chunks = N // K
    iters = (n_chunks + NW - 1) // NW
    rounds = (iters + nbuf - 1) // nbuf
    n_t = len(pairs)
    uidx, uslot = [], []
    for _, ix in pairs:
        for u, ux in enumerate(uidx):
            if ux is ix:
                uslot.append(u)
                break
        else:
            uslot.append(len(uidx))
            uidx.append(ix)
    n_u = len(uidx)
    mesh = plsc.VectorSubcoreMesh(core_axis_name="c", subcore_axis_name="s")

    scratch = (
        [pltpu.VMEM((K,), jnp.int32) for _ in range(n_u * nbuf)]
        + [pltpu.VMEM((K, t.shape[1]), jnp.float32)
           for t, _ in pairs for _ in range(nbuf)]
        + [pltpu.SemaphoreType.DMA for _ in range(2 * nbuf)]
    )

    @functools.partial(
        pl.kernel,
        mesh=mesh,
        out_type=[jax.ShapeDtypeStruct((N, t.shape[1]), jnp.float32)
                  for t, _ in pairs],
        scratch_types=scratch,
    )
    def k(*refs):
        tabs = refs[0:n_t]
        idxs = refs[n_t:n_t + n_u]
        outs = refs[n_t + n_u:n_t + n_u + n_t]
        sc = refs[n_t + n_u + n_t:]
        ib = [sc[u * nbuf:(u + 1) * nbuf] for u in range(n_u)]
        sc = sc[n_u * nbuf:]
        rb = [sc[j * nbuf:(j + 1) * nbuf] for j in range(n_t)]
        sc = sc[n_t * nbuf:]
        gsem, ssem = sc[:nbuf], sc[nbuf:2 * nbuf]
        wid = lax.axis_index("s") * NC + lax.axis_index("c")

        def gath(j, b):
            return pltpu.make_async_copy(
                tabs[j].at[ib[uslot[j]][b]], rb[j][b], gsem[b])

        def stor(j, b, off):
            return pltpu.make_async_copy(
                rb[j][b], outs[j].at[pl.ds(off, K)], ssem[b])

        def body(r, carry):
            cs = []
            for b in range(nbuf):
                cid = (r * nbuf + b) * NW + wid
                cs.append((cid < n_chunks, cid * K))
            for b in range(nbuf):
                pred, off = cs[b]

                @pl.when(pred)
                def _(b=b, off=off):
                    for u in range(n_u):
                        pltpu.sync_copy(idxs[u].at[pl.ds(off, K)], ib[u][b])
                    for j in range(n_t):
                        gath(j, b).start()

            for b in range(nbuf):
                pred, off = cs[b]

                @pl.when(pred)
                def _(b=b, off=off):
                    for j in range(n_t):
                        gath(j, b).wait()
                    for j in range(n_t):
                        stor(j, b, off).start()

            for b in range(nbuf):
                pred, off = cs[b]

                @pl.when(pred)
                def _(b=b, off=off):
                    for j in range(n_t):
                        stor(j, b, off).wait()

            return carry

        lax.fori_loop(0, rounds, body, 0)

    res = k(*[t for t, _ in pairs], *uidx)
    return list(res) if isinstance(res, (list, tuple)) else [res]


def _sc_scatter_multi(vals_list, idx, S):
    N = idx.shape[0]
    n_chunks = N // K
    iters = (n_chunks + NS - 1) // NS
    n_v = len(vals_list)
    zeros = jnp.zeros((S, 128), jnp.float32)
    mesh = plsc.VectorSubcoreMesh(core_axis_name="c", subcore_axis_name="s")
    phases = []
    for v, a in enumerate(vals_list):
        for cb in range(0, a.shape[1], 256):
            phases.append((v, cb))
    r_lo, r_hi = (S // 16) // 8 * 8, S - 15 * ((S // 16) // 8 * 8)

    nbuf = 3
    rounds = (iters + nbuf - 1) // nbuf

    @functools.partial(
        pl.kernel,
        mesh=mesh,
        out_type=[jax.ShapeDtypeStruct((S, a.shape[1]), jnp.float32)
                  for a in vals_list],
        scratch_types=(
            [pltpu.VMEM((K,), jnp.int32) for _ in range(nbuf)]
            + [pltpu.VMEM((K, 128), jnp.float32) for _ in range(nbuf)]
            + [pltpu.VMEM_SHARED((S, 128), jnp.float32)]
            + [pltpu.SemaphoreType.DMA for _ in range(2 * nbuf)]
        ),
    )
    def k(*refs):
        vals = refs[0:n_v]
        idx_hbm = refs[n_v]
        zeros_hbm = refs[n_v + 1]
        outs = refs[n_v + 2:n_v + 2 + n_v]
        sc = refs[n_v + 2 + n_v:]
        ib, vb = sc[:nbuf], sc[nbuf:2 * nbuf]
        acc = sc[2 * nbuf]
        lsem = sc[2 * nbuf + 1:2 * nbuf + 1 + nbuf]
        asem = sc[2 * nbuf + 1 + nbuf:2 * nbuf + 1 + 2 * nbuf]
        c = lax.axis_index("c")
        s = lax.axis_index("s")
        r0 = s * r_lo

        for v, cb in phases:
            D = vals_list[v].shape[1]
            col = cb + c * 128
            active = col < D

            @pl.when(active & (s < 15))
            def _():
                pltpu.sync_copy(zeros_hbm.at[pl.ds(r0, r_lo)],
                                acc.at[pl.ds(r0, r_lo)])

            @pl.when(active & (s == 15))
            def _():
                pltpu.sync_copy(zeros_hbm.at[pl.ds(15 * r_lo, r_hi)],
                                acc.at[pl.ds(15 * r_lo, r_hi)])

            plsc.subcore_barrier()

            def ldi(b, off):
                return pltpu.make_async_copy(
                    idx_hbm.at[pl.ds(off, K)], ib[b], lsem[b])

            def ldv(b, off):
                return pltpu.make_async_copy(
                    vals[v].at[pl.ds(off, K), pl.ds(col, 128)], vb[b], lsem[b])

            def addv(b):
                return pltpu.make_async_copy(vb[b], acc.at[ib[b]], asem[b])

            def body(i, carry):
                cs = []
                for b in range(nbuf):
                    cid = (i * nbuf + b) * NS + s
                    cs.append((active & (cid < n_chunks), cid * K))
                for b in range(nbuf):
                    pred, off = cs[b]

                    @pl.when(pred)
                    def _(b=b, off=off):
                        ldi(b, off).start()
                        ldv(b, off).start()

                for b in range(nbuf):
                    pred, off = cs[b]

                    @pl.when(pred)
                    def _(b=b, off=off):
                        ldi(b, off).wait()
                        ldv(b, off).wait()
                        addv(b).start(add=True)

                for b in range(nbuf):
                    pred, off = cs[b]

                    @pl.when(pred)
                    def _(b=b):
                        addv(b).wait()

                return carry

            lax.fori_loop(0, rounds, body, 0)
            plsc.subcore_barrier()

            @pl.when(active & (s < 15))
            def _():
                pltpu.sync_copy(acc.at[pl.ds(r0, r_lo)],
                                outs[v].at[pl.ds(r0, r_lo), pl.ds(col, 128)])

            @pl.when(active & (s == 15))
            def _():
                pltpu.sync_copy(acc.at[pl.ds(15 * r_lo, r_hi)],
                                outs[v].at[pl.ds(15 * r_lo, r_hi), pl.ds(col, 128)])

            plsc.subcore_barrier()

    res = k(*vals_list, idx, zeros)
    return list(res) if isinstance(res, (list, tuple)) else [res]


def _mlp_refs(x, wrefs):
    h = x * wrefs[0][...]
    for w in wrefs[1:]:
        h = jnp.maximum(h, 0.0)
        h = jnp.dot(h, w[...], preferred_element_type=jnp.float32)
    return h


def _full(shape):
    return pl.BlockSpec(shape, lambda i: tuple(0 for _ in shape))


def _blk(be, d):
    return pl.BlockSpec((be, d), lambda i: (i, 0))


def _tc_edge(emb, sh, g, ws, Bm, be):
    N, D = g.shape
    nw = len(ws)

    def body(*refs):
        emb_ref, sh_ref, g_ref = refs[:3]
        wrefs = refs[3:3 + nw]
        B_ref = refs[3 + nw]
        out_ref = refs[3 + nw + 1]
        w = _mlp_refs(emb_ref[...], wrefs)
        shB = jnp.dot(sh_ref[...], B_ref[...], preferred_element_type=jnp.float32)
        out_ref[...] = w * g_ref[...] * shB

    return pl.pallas_call(
        body,
        grid=(N // be,),
        in_specs=[_blk(be, 1), _blk(be, 9), _blk(be, D)]
        + [_full(w.shape) for w in ws] + [_full(Bm.shape)],
        out_specs=_blk(be, D),
        out_shape=jax.ShapeDtypeStruct((N, D), jnp.float32),
    )(emb, sh, g, *ws, Bm)


def _tc_edge_packed(emb, sh, g, ws, Bm, be):
    N, H = g.shape
    nw = len(ws)

    def body(*refs):
        emb_ref, sh_ref, g_ref = refs[:3]
        wrefs = refs[3:3 + nw]
        B_ref = refs[3 + nw]
        out_ref = refs[3 + nw + 1]
        w = _mlp_refs(emb_ref[...], wrefs)
        shB = jnp.dot(sh_ref[...], B_ref[...], preferred_element_type=jnp.float32)
        m = w * shB
        glo, ghi = _unpack_lo_hi(g_ref[...])
        out_ref[...] = jnp.concatenate([glo * m[:, :H], ghi * m[:, H:]], axis=1)

    return pl.pallas_call(
        body,
        grid=(N // be,),
        in_specs=[_blk(be, 1), _blk(be, 9), _blk(be, H)]
        + [_full(w.shape) for w in ws] + [_full(Bm.shape)],
        out_specs=_blk(be, 2 * H),
        out_shape=jax.ShapeDtypeStruct((N, 2 * H), jnp.float32),
    )(emb, sh, g, *ws, Bm)


def _tc_node_mm(x, mats, br):
    S, Din = x.shape

    def body(*refs):
        x_ref = refs[0]
        m_refs = refs[1:1 + len(mats)]
        out_refs = refs[1 + len(mats):]
        xv = x_ref[...]
        for m_ref, o_ref in zip(m_refs, out_refs):
            o_ref[...] = jnp.dot(xv, m_ref[...], preferred_element_type=jnp.float32)

    return pl.pallas_call(
        body,
        grid=(S // br,),
        in_specs=[_blk(br, Din)] + [_full(m.shape) for m in mats],
        out_specs=[_blk(br, m.shape[1]) for m in mats],
        out_shape=[jax.ShapeDtypeStruct((S, m.shape[1]), jnp.float32) for m in mats],
    )(x, *mats)


def _bf16_round(u):
    return (u + 0x7FFF + ((u >> 16) & 1)) >> 16


def _tc_node_mm_pack(x, M, br):
    S, Din = x.shape
    H = M.shape[1] // 2

    def body(x_ref, m_ref, o_ref):
        t = jnp.dot(x_ref[...], m_ref[...], preferred_element_type=jnp.float32)
        ul = _bf16_round(lax.bitcast_convert_type(t[:, :H], jnp.uint32))
        uh = _bf16_round(lax.bitcast_convert_type(t[:, H:], jnp.uint32))
        o_ref[...] = lax.bitcast_convert_type(ul | (uh << 16), jnp.float32)

    return pl.pallas_call(
        body,
        grid=(S // br,),
        in_specs=[_blk(br, Din), _full(M.shape)],
        out_specs=_blk(br, H),
        out_shape=jax.ShapeDtypeStruct((S, H), jnp.float32),
    )(x, M)


def _unpack_lo_hi(g):
    u = lax.bitcast_convert_type(g, jnp.uint32)
    lo = lax.bitcast_convert_type(u << 16, jnp.float32)
    hi = lax.bitcast_convert_type(u & jnp.uint32(0xFFFF0000), jnp.float32)
    return lo, hi


def _tc_edge_ch(emb, sh, g, wsC, wsH, BC, BH, be):
    N, H = g.shape

    def body(*refs):
        emb_ref, sh_ref, g_ref = refs[:3]
        i = 3
        wC = refs[i:i + 4]; i += 4
        wH = refs[i:i + 4]; i += 4
        BC_ref, BH_ref, lo_ref, hi_ref = refs[i:i + 4]
        shv = sh_ref[...]
        ev = emb_ref[...]
        mC = _mlp_refs(ev, wC)
        mH = _mlp_refs(ev, wH)
        shBC = jnp.dot(shv, BC_ref[...], preferred_element_type=jnp.float32)
        shBH = jnp.dot(shv, BH_ref[...], preferred_element_type=jnp.float32)
        m = mC * shBC + mH * shBH
        glo, ghi = _unpack_lo_hi(g_ref[...])
        lo_ref[...] = glo * m[:, :H]
        hi_ref[...] = ghi * m[:, H:]

    return pl.pallas_call(
        body,
        grid=(N // be,),
        in_specs=[_blk(be, 1), _blk(be, 9), _blk(be, H)]
        + [_full(w.shape) for w in wsC] + [_full(w.shape) for w in wsH]
        + [_full(BC.shape), _full(BH.shape)],
        out_specs=[_blk(be, H), _blk(be, H)],
        out_shape=[jax.ShapeDtypeStruct((N, H), jnp.float32),
                   jax.ShapeDtypeStruct((N, H), jnp.float32)],
    )(emb, sh, g, *wsC, *wsH, BC, BH)


def _tc_bond(emb_b, sh_b, ga, gb, ws_bond, ws_x, Ax, Bx, blk0, nblk, be):
    Dx = Ax.shape[1]
    nb, nx = len(ws_bond), len(ws_x)

    def off(d):
        return pl.BlockSpec((be, d), lambda i: (blk0 + i, 0))

    def body(*refs):
        emb_ref, sh_ref, ga_ref, gb_ref = refs[:4]
        i = 4
        wb = refs[i:i + nb]; i += nb
        wx = refs[i:i + nx]; i += nx
        A_ref, B_ref, out_ref, out2_ref = refs[i:i + 4]
        ev = emb_ref[...]
        ga_lo, ga_hi = _unpack_lo_hi(ga_ref[...])
        gb_lo, gb_hi = _unpack_lo_hi(gb_ref[...])
        wbv = _mlp_refs(ev, wb)
        hf_lo = wbv[:, :128] * ga_lo * gb_lo
        hf_hi = wbv[:, 128:] * ga_hi * gb_hi
        mx = _mlp_refs(ev, wx)
        Av = A_ref[...]
        hA = (jnp.dot(hf_lo, Av[:128], preferred_element_type=jnp.float32)
              + jnp.dot(hf_hi, Av[128:], preferred_element_type=jnp.float32))
        sB = jnp.dot(sh_ref[...], B_ref[...], preferred_element_type=jnp.float32)
        res = mx * hA * sB
        out_ref[...] = res[:, :Dx // 2]
        out2_ref[...] = res[:, Dx // 2:]

    h = Dx // 2
    return pl.pallas_call(
        body,
        grid=(nblk,),
        in_specs=[off(1), off(9), off(ga.shape[1]), off(gb.shape[1])]
        + [_full(w.shape) for w in ws_bond] + [_full(w.shape) for w in ws_x]
        + [_full(Ax.shape), _full(Bx.shape)],
        out_specs=[_blk(be, h), _blk(be, h)],
        out_shape=[jax.ShapeDtypeStruct((nblk * be, h), jnp.float32),
                   jax.ShapeDtypeStruct((nblk * be, h), jnp.float32)],
    )(emb_b, sh_b, ga, gb, *ws_bond, *ws_x, Ax, Bx)


def kernel(sh, emb, f_in, edge_src, edge_dst, num_nodes, num_neighbors,
           HH_ind, CC_ind, CH_ind, fc1, fc2, fc_bond, fcHH, fcCC, fcCH, fcC, fcH,
           A1, B1, A2, B2, Ab, Bb, AHH, BHH, ACC, BCC, ACH, BCH, AC, BC, AH, BH):
    S = f_in.shape[0]
    E = sh.shape[0]
    NB = HH_ind.shape[0]
    NBP = 122880

    inv = 1.0 / jnp.sqrt(jnp.asarray(num_neighbors, jnp.float32))
    w1l = _pad2(fc1[-1] * inv, 16, 128)
    w2l = _pad2(fc2[-1] * inv, 16, 256)
    wCl = _pad2(fcC[-1] * inv, 16, 512)
    wHl = jnp.pad(fcH[-1] * inv, ((0, 0), (392, 70)))
    BHe = jnp.pad(BH, ((0, 0), (392, 70)))

    srcf = edge_src.astype(jnp.float32).reshape(E, 1)
    dstf = edge_dst.astype(jnp.float32).reshape(E, 1)
    meta = jnp.pad(jnp.concatenate([srcf, dstf, emb, sh], axis=1),
                   ((0, 0), (0, 116)))
    inds = jnp.concatenate(
        [HH_ind, CC_ind, CH_ind, jnp.zeros((E - 3 * NB,), jnp.int32)])

    (p1,) = _tc_node_mm(f_in, [_pad2(A1, 2, 128)], 2000)
    (g1,) = _sc_gather_multi([(p1, edge_src)], nbuf=3)
    ef1 = _tc_edge(emb, sh, g1, list(fc1[:-1]) + [w1l], _pad2(B1, 9, 128), 1600)
    (nf1,) = _sc_scatter_multi([ef1], edge_dst, S)

    p2 = _tc_node_mm_pack(nf1, _pad2(A2, 128, 256), 2000)
    (g2,) = _sc_gather_multi([(p2, edge_src)], nbuf=3)
    ef2 = _tc_edge_packed(emb, sh, g2, list(fc2[:-1]) + [w2l],
                          _pad2(B2, 9, 256), 1600)
    (nf2,) = _sc_scatter_multi([ef2], edge_dst, S)

    ACHw = _pad2(jnp.concatenate([AC, AH], axis=1), 256, 512)
    TA = _tc_node_mm_pack(nf2, _pad2(Ab, 256, 256), 2000)
    TB = _tc_node_mm_pack(nf2, _pad2(Bb, 256, 256), 2000)
    TCHt = _tc_node_mm_pack(nf2, ACHw, 2000)

    (gCH,) = _sc_gather_multi([(TCHt, edge_src)], nbuf=3)
    (metag,) = _sc_gather_multi([(meta, inds[:NBP])], nbuf=4)
    eCH_lo, eCH_hi = _tc_edge_ch(
        emb, sh, gCH,
        list(fcC[:-1]) + [wCl], list(fcH[:-1]) + [wHl],
        _pad2(BC, 9, 512), BHe, 1600)
    n_lo, n_hi = _sc_scatter_multi([eCH_lo, eCH_hi], edge_dst, S)
    node_C = jnp.concatenate([n_lo, n_hi[:, :136]], axis=1)
    node_H = n_hi[:, 136:186]

    b_src = metag[:NBP, 0].astype(jnp.int32)
    b_dst = metag[:NBP, 1].astype(jnp.int32)
    emb_b = metag[:NBP, 2:3]
    sh_b = metag[:NBP, 3:12]
    gA, gB = _sc_gather_multi([(TA, b_src), (TB, b_dst)], nbuf=3)

    be = 2000
    nblk = NB // be
    wsb = list(fc_bond[:-1]) + [_pad2(fc_bond[-1], 16, 256)]
    eHH0, eHH1 = _tc_bond(emb_b, sh_b, gA, gB, wsb, fcHH,
                          _pad2(AHH, 256, 50), BHH, 0 * nblk, nblk, be)
    eCC0, eCC1 = _tc_bond(emb_b, sh_b, gA, gB, wsb, fcCC,
                          _pad2(ACC, 256, 392), BCC, 1 * nblk, nblk, be)
    eCH0, eCH1 = _tc_bond(emb_b, sh_b, gA, gB, wsb, fcCH,
                          _pad2(ACH, 256, 140), BCH, 2 * nblk, nblk, be)

    hC, hH = 196, 25
    return (node_H[:, :hH], node_C[:, :hC], eHH0, eCH0, eCC0,
            node_H[:, hH:2 * hH], node_C[:, hC:2 * hC], eHH1, eCH1, eCC1)
